# baseline (speedup 1.0000x reference)
import functools
import math

import jax
import jax.numpy as jnp
from jax import lax
from jax.experimental import pallas as pl
from jax.experimental.pallas import tpu as pltpu
from jax.experimental.pallas import tpu_sc as plsc

C = 128
NUM_HEADS, HEAD_DIM = 8, 16


def _ln(xb, g, b, eps=1e-5):
    mu = jnp.mean(xb, axis=-1, keepdims=True)
    var = jnp.mean((xb - mu) ** 2, axis=-1, keepdims=True)
    return (xb - mu) / jnp.sqrt(var + eps) * g + b


def _n1_body(x_ref, tbl_ref, pg, pb, qwt, qb, kwt, vwt, vb,
             xn_ref, q_ref, kv_ref):
    xb = x_ref[...]
    tbl = tbl_ref[...]
    pad = jnp.zeros((xb.shape[0], 112), jnp.float32)
    xn = _ln(xb, pg[...], pb[...])
    xn_ref[...] = xn
    q = jnp.dot(xn, qwt[...], preferred_element_type=jnp.float32) + qb[...]
    q_ref[...] = jnp.concatenate([q, tbl, pad], axis=-1)
    k = jnp.dot(xn, kwt[...], preferred_element_type=jnp.float32)
    v = jnp.dot(xn, vwt[...], preferred_element_type=jnp.float32) + vb[...]
    kv_ref[...] = jnp.concatenate([k, v, tbl, pad], axis=-1)


_TWO_PI = 2.0 * math.pi
_INV_2PI = 1.0 / _TWO_PI
_SIN_C = (1.0, -1.0 / 6, 1.0 / 120, -1.0 / 5040, 1.0 / 362880,
          -1.0 / 39916800, 1.0 / 6227020800)
_COS_C = (1.0, -0.5, 1.0 / 24, -1.0 / 720, 1.0 / 40320, -1.0 / 3628800,
          1.0 / 479001600)


def _e0_body(ts_ref, td_ref, out_ref):
    ts = ts_ref[...]
    td = td_ref[...]
    relx = ts[0:1, :] - td[0:1, :]
    rely = ts[1:2, :] - td[1:2, :]
    d2 = relx * relx + rely * rely + 1e-12
    dist = d2 * lax.rsqrt(d2)
    cx = td[2:3, :]
    cy = td[3:4, :]
    direction = jnp.arctan2(cx * rely - cy * relx, cx * relx + cy * rely)
    ha = ts[4:5, :] - td[4:5, :]
    rh = ha - _TWO_PI * _round_ne(ha * _INV_2PI)
    out_ref[...] = jnp.concatenate(
        [dist, direction, rh,
         jnp.zeros((5, ts.shape[1]), jnp.float32)], axis=0)


def _tc_e0(tsT, tdT, bt):
    e = tsT.shape[1]
    grid = e // bt
    return pl.pallas_call(
        _e0_body,
        grid=(grid,),
        in_specs=[pl.BlockSpec((16, bt), lambda i: (0, i)),
                  pl.BlockSpec((16, bt), lambda i: (0, i))],
        out_specs=[pl.BlockSpec((8, bt), lambda i: (0, i))],
        out_shape=[jax.ShapeDtypeStruct((8, e), jnp.float32)],
    )(tsT, tdT)[0]


def _poly_even(y2, coefs):
    acc = jnp.full_like(y2, coefs[-1])
    for cc in coefs[-2::-1]:
        acc = acc * y2 + cc
    return acc


def _round_ne(x):
    return jnp.round(x)


def _e1_body(qd_ref, kv_ref, rel8_ref,
             f2p, w1cc, w1cs, w1r, b1, g1, be1, w2t, b2,
             olng, olnb, outwt, outb, rg, rb, krt, vrt, vrb, s16,
             sim_ref, vj_ref, bmax_ref):
    qde = qd_ref[...]
    kve = kv_ref[...]
    rel8 = rel8_ref[...]
    xfa = jnp.concatenate(
        [rel8[:, i:i + 1] * f2p[i:i + 1, :] for i in range(3)], axis=-1)
    y = xfa - _TWO_PI * _round_ne(xfa * _INV_2PI)
    y2 = y * y
    sina = y * _poly_even(y2, _SIN_C)
    cosa = _poly_even(y2, _COS_C)
    acc = jnp.zeros((qde.shape[0], C), jnp.float32)
    for i in range(3):
        h = (jnp.dot(cosa[:, 64 * i:64 * (i + 1)], w1cc[i],
                     preferred_element_type=jnp.float32)
             + jnp.dot(sina[:, 64 * i:64 * (i + 1)], w1cs[i],
                       preferred_element_type=jnp.float32)
             + rel8[:, i:i + 1] * w1r[i:i + 1, :] + b1[i:i + 1, :])
        h = _ln(h, g1[i:i + 1, :], be1[i:i + 1, :])
        h = jax.nn.relu(h)
        acc = acc + jnp.dot(h, w2t[i], preferred_element_type=jnp.float32) + b2[i:i + 1, :]
    r = jax.nn.relu(_ln(acc, olng[...], olnb[...]))
    r = jnp.dot(r, outwt[...], preferred_element_type=jnp.float32) + outb[...]
    rn = _ln(r, rg[...], rb[...])
    kj = kve[:, :C] + jnp.dot(rn, krt[...], preferred_element_type=jnp.float32)
    vj = (kve[:, C:2 * C]
          + jnp.dot(rn, vrt[...], preferred_element_type=jnp.float32) + vrb[...])
    sim = jnp.dot(qde[:, :C] * kj, s16[...],
                  preferred_element_type=jnp.float32) * (HEAD_DIM ** -0.5)
    sim_ref[...] = sim
    vj_ref[...] = vj
    bmax_ref[...] = jnp.max(sim, axis=0, keepdims=True)[None]


def _e2_body(sim_ref, vj_ref, gmax, s16t, wv_ref, wp_ref):
    w = jnp.exp(sim_ref[...] - gmax[...])
    wv_ref[...] = vj_ref[...] * jnp.dot(w, s16t[...],
                                        preferred_element_type=jnp.float32)
    wp_ref[...] = jnp.concatenate(
        [w, jnp.zeros((w.shape[0], C - NUM_HEADS), jnp.float32)], axis=-1)


def _n2_body(x_ref, xn_ref, aggp_ref, sp_ref, s16t,
             gwt1, gwt2, gb, swt, sb, owt, ob, postg, postb,
             ffpreg, ffpreb, ffw1t, ffb1, ffw2t, ffb2, ffpostg, ffpostb,
             out_ref):
    xb = x_ref[...]
    xn = xn_ref[...]
    ap = aggp_ref[...]
    sp = sp_ref[...]
    aggs = ap[0] + ap[1]
    ss = sp[0][:, 0:NUM_HEADS] + sp[1][:, 0:NUM_HEADS]
    recip = 1.0 / (ss + 1e-16)
    agg = aggs * jnp.dot(recip, s16t[...], preferred_element_type=jnp.float32)
    g = jax.nn.sigmoid(
        jnp.dot(agg, gwt1[...], preferred_element_type=jnp.float32)
        + jnp.dot(xn, gwt2[...], preferred_element_type=jnp.float32) + gb[...])
    sk = jnp.dot(xn, swt[...], preferred_element_type=jnp.float32) + sb[...]
    agg = agg + g * (sk - agg)
    out = jnp.dot(agg, owt[...], preferred_element_type=jnp.float32) + ob[...]
    x1 = xb + _ln(out, postg[...], postb[...])
    h = _ln(x1, ffpreg[...], ffpreb[...])
    h = jax.nn.relu(jnp.dot(h, ffw1t[...], preferred_element_type=jnp.float32)
                    + ffb1[...])
    h = jnp.dot(h, ffw2t[...], preferred_element_type=jnp.float32) + ffb2[...]
    out_ref[...] = x1 + _ln(h, ffpostg[...], ffpostb[...])


def _row(v):
    return v.reshape(1, -1)


def _pad_rows(a, rows=8):
    return jnp.pad(a, ((0, rows - a.shape[0]), (0, 0)))


def _full_spec(shape):
    nd = len(shape)
    return pl.BlockSpec(shape, lambda i, _nd=nd: (0,) * _nd)


def _tc_n1(xf, tbl, at, bn):
    n = xf.shape[0]
    grid = n // bn
    specs = [pl.BlockSpec((bn, C), lambda i: (i, 0)),
             pl.BlockSpec((bn, 16), lambda i: (i, 0))]
    wspecs = [_full_spec((1, C)), _full_spec((1, C)), _full_spec((C, C)),
              _full_spec((1, C)), _full_spec((C, C)), _full_spec((C, C)),
              _full_spec((1, C))]
    return pl.pallas_call(
        _n1_body,
        grid=(grid,),
        in_specs=specs + wspecs,
        out_specs=[pl.BlockSpec((bn, C), lambda i: (i, 0)),
                   pl.BlockSpec((bn, 2 * C), lambda i: (i, 0)),
                   pl.BlockSpec((bn, 3 * C), lambda i: (i, 0))],
        out_shape=[jax.ShapeDtypeStruct((n, C), jnp.float32),
                   jax.ShapeDtypeStruct((n, 2 * C), jnp.float32),
                   jax.ShapeDtypeStruct((n, 3 * C), jnp.float32)],
    )(xf, tbl, _row(at['pre_g']), _row(at['pre_b']), at['q_w'].T,
      _row(at['q_b']), at['k_w'].T, at['v_w'].T, _row(at['v_b']))


def _tc_e1(qd, kvs, rel8, fe, at, s16, be):
    e = qd.shape[0]
    grid = e // be
    f2p = _pad_rows(fe['freqs'] * (2 * math.pi))
    w1cc = jnp.transpose(fe['mlp_w1'][:, :, :64], (0, 2, 1))
    w1cs = jnp.transpose(fe['mlp_w1'][:, :, 64:C], (0, 2, 1))
    w1r = _pad_rows(fe['mlp_w1'][:, :, C])
    w2t = jnp.transpose(fe['mlp_w2'], (0, 2, 1))
    espec = lambda w: pl.BlockSpec((be, w), lambda i: (i, 0))
    in_specs = [espec(2 * C), espec(3 * C), espec(8),
                _full_spec((8, 64)), _full_spec((3, 64, C)),
                _full_spec((3, 64, C)), _full_spec((8, C)),
                _full_spec((8, C)), _full_spec((8, C)), _full_spec((8, C)),
                _full_spec((3, C, C)), _full_spec((8, C)),
                _full_spec((1, C)), _full_spec((1, C)), _full_spec((C, C)),
                _full_spec((1, C)), _full_spec((1, C)), _full_spec((1, C)),
                _full_spec((C, C)), _full_spec((C, C)), _full_spec((1, C)),
                _full_spec((C, NUM_HEADS))]
    return pl.pallas_call(
        _e1_body,
        grid=(grid,),
        in_specs=in_specs,
        out_specs=[espec(NUM_HEADS), espec(C),
                   pl.BlockSpec((1, 1, NUM_HEADS), lambda i: (i, 0, 0))],
        out_shape=[jax.ShapeDtypeStruct((e, NUM_HEADS), jnp.float32),
                   jax.ShapeDtypeStruct((e, C), jnp.float32),
                   jax.ShapeDtypeStruct((grid, 1, NUM_HEADS), jnp.float32)],
    )(qd, kvs, rel8, f2p, w1cc, w1cs, w1r, _pad_rows(fe['mlp_b1']),
      _pad_rows(fe['mlp_ln_g']), _pad_rows(fe['mlp_ln_b']), w2t,
      _pad_rows(fe['mlp_b2']), _row(fe['out_ln_g']), _row(fe['out_ln_b']),
      fe['out_w'].T, _row(fe['out_b']), _row(at['r_g']), _row(at['r_b']),
      at['kr_w'].T, at['vr_w'].T, _row(at['vr_b']), s16)


def _tc_e2(sim, vj, gmax, s16t, be):
    e = sim.shape[0]
    grid = e // be
    return pl.pallas_call(
        _e2_body,
        grid=(grid,),
        in_specs=[pl.BlockSpec((be, NUM_HEADS), lambda i: (i, 0)),
                  pl.BlockSpec((be, C), lambda i: (i, 0)),
                  _full_spec((1, NUM_HEADS)), _full_spec((NUM_HEADS, C))],
        out_specs=[pl.BlockSpec((be, C), lambda i: (i, 0)),
                   pl.BlockSpec((be, C), lambda i: (i, 0))],
        out_shape=[jax.ShapeDtypeStruct((e, C), jnp.float32),
                   jax.ShapeDtypeStruct((e, C), jnp.float32)],
    )(sim, vj, gmax, s16t)


def _tc_n2(xf, xn, aggp, sp, at, s16t, bn):
    n = xf.shape[0]
    grid = n // bn
    nspec = lambda w: pl.BlockSpec((bn, w), lambda i: (i, 0))
    return pl.pallas_call(
        _n2_body,
        grid=(grid,),
        in_specs=[nspec(C), nspec(C),
                  pl.BlockSpec((2, bn, C), lambda i: (0, i, 0)),
                  pl.BlockSpec((2, bn, C), lambda i: (0, i, 0)),
                  _full_spec((NUM_HEADS, C)),
                  _full_spec((C, C)), _full_spec((C, C)), _full_spec((1, C)),
                  _full_spec((C, C)), _full_spec((1, C)),
                  _full_spec((C, C)), _full_spec((1, C)),
                  _full_spec((1, C)), _full_spec((1, C)),
                  _full_spec((1, C)), _full_spec((1, C)),
                  _full_spec((C, 4 * C)), _full_spec((1, 4 * C)),
                  _full_spec((4 * C, C)), _full_spec((1, C)),
                  _full_spec((1, C)), _full_spec((1, C))],
        out_specs=[nspec(C)],
        out_shape=[jax.ShapeDtypeStruct((n, C), jnp.float32)],
    )(xf, xn, aggp, sp, s16t,
      at['g_w'][:, :C].T, at['g_w'][:, C:].T, _row(at['g_b']),
      at['s_w'].T, _row(at['s_b']), at['o_w'].T, _row(at['o_b']),
      _row(at['post_g']), _row(at['post_b']),
      _row(at['ffpre_g']), _row(at['ffpre_b']),
      at['ff_w1'].T, _row(at['ff_b1']), at['ff_w2'].T, _row(at['ff_b2']),
      _row(at['ffpost_g']), _row(at['ffpost_b']))[0]


def _sc_gather(src, dst, q, kv):
    e = src.shape[0]
    info = plsc.get_sparse_core_info()
    nw = info.num_cores * info.num_subcores
    epw = e // nw
    ch = 144
    assert epw % ch == 0
    nit = epw // ch
    mesh = plsc.VectorSubcoreMesh(core_axis_name="c", subcore_axis_name="s")

    @functools.partial(
        pl.kernel, mesh=mesh,
        out_type=[jax.ShapeDtypeStruct((e, 2 * C), jnp.float32),
                  jax.ShapeDtypeStruct((e, 3 * C), jnp.float32)],
        scratch_types=[pltpu.VMEM((ch,), jnp.int32),
                       pltpu.VMEM((ch,), jnp.int32),
                       pltpu.VMEM((ch, 2 * C), jnp.float32),
                       pltpu.VMEM((ch, 3 * C), jnp.float32),
                       pltpu.SemaphoreType.DMA],
    )
    def g1(src_h, dst_h, q_h, kv_h, qd_o, kvs_o, sidx, didx, qv, kvv, sem):
        wid = lax.axis_index("s") * info.num_cores + lax.axis_index("c")
        base = wid * epw

        def body(it, carry):
            off = base + it * ch
            pltpu.sync_copy(src_h.at[pl.ds(off, ch)], sidx)
            pltpu.sync_copy(dst_h.at[pl.ds(off, ch)], didx)
            c1 = pltpu.async_copy(q_h.at[didx], qv, sem)
            c2 = pltpu.async_copy(kv_h.at[sidx], kvv, sem)
            c1.wait()
            c2.wait()
            pltpu.sync_copy(qv, qd_o.at[pl.ds(off, ch)])
            pltpu.sync_copy(kvv, kvs_o.at[pl.ds(off, ch)])
            return carry

        lax.fori_loop(0, nit, body, 0)

    return g1(src, dst, q, kv)


def _sc_scatter(dst, wv, wp, n):
    e = dst.shape[0]
    info = plsc.get_sparse_core_info()
    nw = info.num_cores * info.num_subcores
    epw = e // nw
    ch = 336
    assert epw % ch == 0
    nit = epw // ch
    rpt = -(-n // info.num_subcores)
    rpt += (-rpt) % 8
    rlast = n - (info.num_subcores - 1) * rpt
    assert rlast > 0
    mesh = plsc.VectorSubcoreMesh(core_axis_name="c", subcore_axis_name="s")
    zeros = jnp.zeros((n, C), jnp.float32)

    @functools.partial(
        pl.kernel, mesh=mesh,
        out_type=[jax.ShapeDtypeStruct((2, n, C), jnp.float32),
                  jax.ShapeDtypeStruct((2, n, C), jnp.float32)],
        scratch_types=[pltpu.VMEM((ch,), jnp.int32),
                       pltpu.VMEM((ch, C), jnp.float32),
                       pltpu.VMEM_SHARED((n, C), jnp.float32)],
    )
    def s1(dst_h, wv_h, wp_h, z_h, aggp_o, sp_o, idxv, rows, acc_sh):
        cid = lax.axis_index("c")
        sid = lax.axis_index("s")
        wid = sid * info.num_cores + cid
        base = wid * epw

        def phase(val_h, out_h):
            @pl.when(sid == 0)
            def _init():
                pltpu.sync_copy(z_h, acc_sh)

            plsc.subcore_barrier()

            def body(it, carry):
                off = base + it * ch
                pltpu.sync_copy(dst_h.at[pl.ds(off, ch)], idxv)
                pltpu.sync_copy(val_h.at[pl.ds(off, ch)], rows)
                pltpu.sync_copy(rows, acc_sh.at[idxv], add=True)
                return carry

            lax.fori_loop(0, nit, body, 0)
            plsc.subcore_barrier()

            @pl.when(sid < info.num_subcores - 1)
            def _copy_main():
                off = pl.multiple_of(sid * rpt, 8)
                pltpu.sync_copy(acc_sh.at[pl.ds(off, rpt)],
                                out_h.at[cid, pl.ds(off, rpt)])

            @pl.when(sid == info.num_subcores - 1)
            def _copy_tail():
                off = pl.multiple_of((info.num_subcores - 1) * rpt, 8)
                pltpu.sync_copy(acc_sh.at[pl.ds(off, rlast)],
                                out_h.at[cid, pl.ds(off, rlast)])

            plsc.subcore_barrier()

        phase(wv_h, aggp_o)
        phase(wp_h, sp_o)

    return s1(dst, wv, wp, zeros)


def kernel(x, pos, head, edges, params):
    a, m, t, c = x.shape
    n = a * m * t
    xf = jnp.transpose(x, (2, 1, 0, 3)).reshape(n, c)
    posf = jnp.transpose(pos, (2, 1, 0, 3)).reshape(n, 2)
    headf = jnp.transpose(head, (2, 1, 0)).reshape(n, 1)
    src = edges[0].astype(jnp.int32)
    dst = edges[1].astype(jnp.int32)
    fe, at = params['fe'], params['attn']

    tbl = jnp.concatenate(
        [posf, jnp.cos(headf), jnp.sin(headf), headf,
         jnp.zeros((n, 11), jnp.float32)], axis=-1)

    s16 = (jnp.arange(c)[:, None] // HEAD_DIM
           == jnp.arange(NUM_HEADS)[None, :]).astype(jnp.float32)
    s16t = s16.T

    xn, q, kv = _tc_n1(xf, tbl, at, bn=1008)
    qd, kvs = _sc_gather(src, dst, q, kv)
    e = qd.shape[0]
    tsT = lax.slice(kvs, (0, 2 * c), (e, 2 * c + 16)).T
    tdT = lax.slice(qd, (0, c), (e, c + 16)).T
    rel8 = _tc_e0(tsT, tdT, bt=13440).T
    sim, vj, bmax = _tc_e1(qd, kvs, rel8, fe, at, s16, be=1344)
    gmax = jnp.max(bmax, axis=0)
    wv, wp = _tc_e2(sim, vj, gmax, s16t, be=2688)
    aggp, sp = _sc_scatter(dst, wv, wp, n)
    out = _tc_n2(xf, xn, aggp, sp, at, s16t, bn=1008)
    return jnp.transpose(out.reshape(t, m, a, c), (2, 1, 0, 3))


# speedup vs baseline: 1.1240x; 1.1240x over previous
import functools
import math

import jax
import jax.numpy as jnp
from jax import lax
from jax.experimental import pallas as pl
from jax.experimental.pallas import tpu as pltpu
from jax.experimental.pallas import tpu_sc as plsc

C = 128
NUM_HEADS, HEAD_DIM = 8, 16


def _ln(xb, g, b, eps=1e-5):
    mu = jnp.mean(xb, axis=-1, keepdims=True)
    var = jnp.mean((xb - mu) ** 2, axis=-1, keepdims=True)
    return (xb - mu) / jnp.sqrt(var + eps) * g + b


def _n1_body(x_ref, tbl_ref, pg, pb, qwt, qb, kwt, vwt, vb,
             xn_ref, q_ref, kv_ref):
    xb = x_ref[...]
    tbl = tbl_ref[...]
    pad = jnp.zeros((xb.shape[0], 112), jnp.float32)
    xn = _ln(xb, pg[...], pb[...])
    xn_ref[...] = xn
    q = jnp.dot(xn, qwt[...], preferred_element_type=jnp.float32) + qb[...]
    q_ref[...] = jnp.concatenate([q, tbl, pad], axis=-1)
    k = jnp.dot(xn, kwt[...], preferred_element_type=jnp.float32)
    v = jnp.dot(xn, vwt[...], preferred_element_type=jnp.float32) + vb[...]
    kv_ref[...] = jnp.concatenate([k, v, tbl, pad], axis=-1)


_TWO_PI = 2.0 * math.pi
_INV_2PI = 1.0 / _TWO_PI
_SIN_C = (1.0, -1.0 / 6, 1.0 / 120, -1.0 / 5040, 1.0 / 362880,
          -1.0 / 39916800, 1.0 / 6227020800)
_COS_C = (1.0, -0.5, 1.0 / 24, -1.0 / 720, 1.0 / 40320, -1.0 / 3628800,
          1.0 / 479001600)


def _poly_even(y2, coefs):
    acc = jnp.full_like(y2, coefs[-1])
    for cc in coefs[-2::-1]:
        acc = acc * y2 + cc
    return acc


def _round_ne(x):
    return jnp.round(x)


def _e1_body(qd_ref, kv_ref,
             f2p, w1cc, w1cs, w1r, b1, g1, be1, w2t, b2,
             olng, olnb, outwt, outb, rg, rb, krt, vrt, vrb, s16,
             sim_ref, vj_ref, bmax_ref):
    qde = qd_ref[...]
    kve = kv_ref[...]
    ts = kve[:, 2 * C:2 * C + 16]
    td = qde[:, C:C + 16]
    relx = ts[:, 0:1] - td[:, 0:1]
    rely = ts[:, 1:2] - td[:, 1:2]
    d2 = relx * relx + rely * rely + 1e-12
    dist = d2 * lax.rsqrt(d2)
    cx = td[:, 2:3]
    cy = td[:, 3:4]
    direction = jnp.arctan2(cx * rely - cy * relx, cx * relx + cy * rely)
    ha = ts[:, 4:5] - td[:, 4:5]
    rh = ha - _TWO_PI * _round_ne(ha * _INV_2PI)
    rel = (dist, direction, rh)
    xfa = jnp.concatenate(
        [rel[i] * f2p[i:i + 1, :] for i in range(3)], axis=-1)
    y = xfa - _TWO_PI * _round_ne(xfa * _INV_2PI)
    y2 = y * y
    sina = y * _poly_even(y2, _SIN_C)
    cosa = _poly_even(y2, _COS_C)
    acc = jnp.zeros((qde.shape[0], C), jnp.float32)
    for i in range(3):
        h = (jnp.dot(cosa[:, 64 * i:64 * (i + 1)], w1cc[i],
                     preferred_element_type=jnp.float32)
             + jnp.dot(sina[:, 64 * i:64 * (i + 1)], w1cs[i],
                       preferred_element_type=jnp.float32)
             + rel[i] * w1r[i:i + 1, :] + b1[i:i + 1, :])
        h = _ln(h, g1[i:i + 1, :], be1[i:i + 1, :])
        h = jax.nn.relu(h)
        acc = acc + jnp.dot(h, w2t[i], preferred_element_type=jnp.float32) + b2[i:i + 1, :]
    r = jax.nn.relu(_ln(acc, olng[...], olnb[...]))
    r = jnp.dot(r, outwt[...], preferred_element_type=jnp.float32) + outb[...]
    rn = _ln(r, rg[...], rb[...])
    kj = kve[:, :C] + jnp.dot(rn, krt[...], preferred_element_type=jnp.float32)
    vj = (kve[:, C:2 * C]
          + jnp.dot(rn, vrt[...], preferred_element_type=jnp.float32) + vrb[...])
    sim = jnp.dot(qde[:, :C] * kj, s16[...],
                  preferred_element_type=jnp.float32) * (HEAD_DIM ** -0.5)
    sim_ref[...] = sim
    vj_ref[...] = vj
    bmax_ref[...] = jnp.max(sim, axis=0, keepdims=True)[None]


def _e2_body(sim_ref, vj_ref, gmax, s16t, wv_ref, wp_ref):
    w = jnp.exp(sim_ref[...] - gmax[...])
    wv_ref[...] = vj_ref[...] * jnp.dot(w, s16t[...],
                                        preferred_element_type=jnp.float32)
    wp_ref[...] = jnp.concatenate(
        [w, jnp.zeros((w.shape[0], C - NUM_HEADS), jnp.float32)], axis=-1)


def _n2_body(x_ref, xn_ref, aggp_ref, sp_ref, s16t,
             gwt1, gwt2, gb, swt, sb, owt, ob, postg, postb,
             ffpreg, ffpreb, ffw1t, ffb1, ffw2t, ffb2, ffpostg, ffpostb,
             out_ref):
    xb = x_ref[...]
    xn = xn_ref[...]
    ap = aggp_ref[...]
    sp = sp_ref[...]
    aggs = ap[0] + ap[1]
    ss = sp[0][:, 0:NUM_HEADS] + sp[1][:, 0:NUM_HEADS]
    recip = 1.0 / (ss + 1e-16)
    agg = aggs * jnp.dot(recip, s16t[...], preferred_element_type=jnp.float32)
    g = jax.nn.sigmoid(
        jnp.dot(agg, gwt1[...], preferred_element_type=jnp.float32)
        + jnp.dot(xn, gwt2[...], preferred_element_type=jnp.float32) + gb[...])
    sk = jnp.dot(xn, swt[...], preferred_element_type=jnp.float32) + sb[...]
    agg = agg + g * (sk - agg)
    out = jnp.dot(agg, owt[...], preferred_element_type=jnp.float32) + ob[...]
    x1 = xb + _ln(out, postg[...], postb[...])
    h = _ln(x1, ffpreg[...], ffpreb[...])
    h = jax.nn.relu(jnp.dot(h, ffw1t[...], preferred_element_type=jnp.float32)
                    + ffb1[...])
    h = jnp.dot(h, ffw2t[...], preferred_element_type=jnp.float32) + ffb2[...]
    out_ref[...] = x1 + _ln(h, ffpostg[...], ffpostb[...])


def _row(v):
    return v.reshape(1, -1)


def _pad_rows(a, rows=8):
    return jnp.pad(a, ((0, rows - a.shape[0]), (0, 0)))


def _full_spec(shape):
    nd = len(shape)
    return pl.BlockSpec(shape, lambda i, _nd=nd: (0,) * _nd)


def _tc_n1(xf, tbl, at, bn):
    n = xf.shape[0]
    grid = n // bn
    specs = [pl.BlockSpec((bn, C), lambda i: (i, 0)),
             pl.BlockSpec((bn, 16), lambda i: (i, 0))]
    wspecs = [_full_spec((1, C)), _full_spec((1, C)), _full_spec((C, C)),
              _full_spec((1, C)), _full_spec((C, C)), _full_spec((C, C)),
              _full_spec((1, C))]
    return pl.pallas_call(
        _n1_body,
        grid=(grid,),
        in_specs=specs + wspecs,
        out_specs=[pl.BlockSpec((bn, C), lambda i: (i, 0)),
                   pl.BlockSpec((bn, 2 * C), lambda i: (i, 0)),
                   pl.BlockSpec((bn, 3 * C), lambda i: (i, 0))],
        out_shape=[jax.ShapeDtypeStruct((n, C), jnp.float32),
                   jax.ShapeDtypeStruct((n, 2 * C), jnp.float32),
                   jax.ShapeDtypeStruct((n, 3 * C), jnp.float32)],
    )(xf, tbl, _row(at['pre_g']), _row(at['pre_b']), at['q_w'].T,
      _row(at['q_b']), at['k_w'].T, at['v_w'].T, _row(at['v_b']))


def _tc_e1(qd, kvs, fe, at, s16, be):
    e = qd.shape[0]
    grid = e // be
    f2p = _pad_rows(fe['freqs'] * (2 * math.pi))
    w1cc = jnp.transpose(fe['mlp_w1'][:, :, :64], (0, 2, 1))
    w1cs = jnp.transpose(fe['mlp_w1'][:, :, 64:C], (0, 2, 1))
    w1r = _pad_rows(fe['mlp_w1'][:, :, C])
    w2t = jnp.transpose(fe['mlp_w2'], (0, 2, 1))
    espec = lambda w: pl.BlockSpec((be, w), lambda i: (i, 0))
    in_specs = [espec(2 * C), espec(3 * C),
                _full_spec((8, 64)), _full_spec((3, 64, C)),
                _full_spec((3, 64, C)), _full_spec((8, C)),
                _full_spec((8, C)), _full_spec((8, C)), _full_spec((8, C)),
                _full_spec((3, C, C)), _full_spec((8, C)),
                _full_spec((1, C)), _full_spec((1, C)), _full_spec((C, C)),
                _full_spec((1, C)), _full_spec((1, C)), _full_spec((1, C)),
                _full_spec((C, C)), _full_spec((C, C)), _full_spec((1, C)),
                _full_spec((C, NUM_HEADS))]
    return pl.pallas_call(
        _e1_body,
        grid=(grid,),
        in_specs=in_specs,
        out_specs=[espec(NUM_HEADS), espec(C),
                   pl.BlockSpec((1, 1, NUM_HEADS), lambda i: (i, 0, 0))],
        out_shape=[jax.ShapeDtypeStruct((e, NUM_HEADS), jnp.float32),
                   jax.ShapeDtypeStruct((e, C), jnp.float32),
                   jax.ShapeDtypeStruct((grid, 1, NUM_HEADS), jnp.float32)],
    )(qd, kvs, f2p, w1cc, w1cs, w1r, _pad_rows(fe['mlp_b1']),
      _pad_rows(fe['mlp_ln_g']), _pad_rows(fe['mlp_ln_b']), w2t,
      _pad_rows(fe['mlp_b2']), _row(fe['out_ln_g']), _row(fe['out_ln_b']),
      fe['out_w'].T, _row(fe['out_b']), _row(at['r_g']), _row(at['r_b']),
      at['kr_w'].T, at['vr_w'].T, _row(at['vr_b']), s16)


def _tc_e2(sim, vj, gmax, s16t, be):
    e = sim.shape[0]
    grid = e // be
    return pl.pallas_call(
        _e2_body,
        grid=(grid,),
        in_specs=[pl.BlockSpec((be, NUM_HEADS), lambda i: (i, 0)),
                  pl.BlockSpec((be, C), lambda i: (i, 0)),
                  _full_spec((1, NUM_HEADS)), _full_spec((NUM_HEADS, C))],
        out_specs=[pl.BlockSpec((be, C), lambda i: (i, 0)),
                   pl.BlockSpec((be, C), lambda i: (i, 0))],
        out_shape=[jax.ShapeDtypeStruct((e, C), jnp.float32),
                   jax.ShapeDtypeStruct((e, C), jnp.float32)],
    )(sim, vj, gmax, s16t)


def _tc_n2(xf, xn, aggp, sp, at, s16t, bn):
    n = xf.shape[0]
    grid = n // bn
    nspec = lambda w: pl.BlockSpec((bn, w), lambda i: (i, 0))
    return pl.pallas_call(
        _n2_body,
        grid=(grid,),
        in_specs=[nspec(C), nspec(C),
                  pl.BlockSpec((2, bn, C), lambda i: (0, i, 0)),
                  pl.BlockSpec((2, bn, C), lambda i: (0, i, 0)),
                  _full_spec((NUM_HEADS, C)),
                  _full_spec((C, C)), _full_spec((C, C)), _full_spec((1, C)),
                  _full_spec((C, C)), _full_spec((1, C)),
                  _full_spec((C, C)), _full_spec((1, C)),
                  _full_spec((1, C)), _full_spec((1, C)),
                  _full_spec((1, C)), _full_spec((1, C)),
                  _full_spec((C, 4 * C)), _full_spec((1, 4 * C)),
                  _full_spec((4 * C, C)), _full_spec((1, C)),
                  _full_spec((1, C)), _full_spec((1, C))],
        out_specs=[nspec(C)],
        out_shape=[jax.ShapeDtypeStruct((n, C), jnp.float32)],
    )(xf, xn, aggp, sp, s16t,
      at['g_w'][:, :C].T, at['g_w'][:, C:].T, _row(at['g_b']),
      at['s_w'].T, _row(at['s_b']), at['o_w'].T, _row(at['o_b']),
      _row(at['post_g']), _row(at['post_b']),
      _row(at['ffpre_g']), _row(at['ffpre_b']),
      at['ff_w1'].T, _row(at['ff_b1']), at['ff_w2'].T, _row(at['ff_b2']),
      _row(at['ffpost_g']), _row(at['ffpost_b']))[0]


def _sc_gather(src, dst, q, kv):
    e = src.shape[0]
    info = plsc.get_sparse_core_info()
    nw = info.num_cores * info.num_subcores
    epw = e // nw
    ch = 144
    assert epw % ch == 0
    nit = epw // ch
    mesh = plsc.VectorSubcoreMesh(core_axis_name="c", subcore_axis_name="s")

    @functools.partial(
        pl.kernel, mesh=mesh,
        out_type=[jax.ShapeDtypeStruct((e, 2 * C), jnp.float32),
                  jax.ShapeDtypeStruct((e, 3 * C), jnp.float32)],
        scratch_types=[pltpu.VMEM((ch,), jnp.int32),
                       pltpu.VMEM((ch,), jnp.int32),
                       pltpu.VMEM((ch, 2 * C), jnp.float32),
                       pltpu.VMEM((ch, 3 * C), jnp.float32),
                       pltpu.SemaphoreType.DMA],
    )
    def g1(src_h, dst_h, q_h, kv_h, qd_o, kvs_o, sidx, didx, qv, kvv, sem):
        wid = lax.axis_index("s") * info.num_cores + lax.axis_index("c")
        base = wid * epw

        def body(it, carry):
            off = base + it * ch
            pltpu.sync_copy(src_h.at[pl.ds(off, ch)], sidx)
            pltpu.sync_copy(dst_h.at[pl.ds(off, ch)], didx)
            c1 = pltpu.async_copy(q_h.at[didx], qv, sem)
            c2 = pltpu.async_copy(kv_h.at[sidx], kvv, sem)
            c1.wait()
            c2.wait()
            pltpu.sync_copy(qv, qd_o.at[pl.ds(off, ch)])
            pltpu.sync_copy(kvv, kvs_o.at[pl.ds(off, ch)])
            return carry

        lax.fori_loop(0, nit, body, 0)

    return g1(src, dst, q, kv)


def _sc_scatter(dst, wv, wp, n):
    e = dst.shape[0]
    info = plsc.get_sparse_core_info()
    nw = info.num_cores * info.num_subcores
    epw = e // nw
    ch = 336
    assert epw % ch == 0
    nit = epw // ch
    rpt = -(-n // info.num_subcores)
    rpt += (-rpt) % 8
    rlast = n - (info.num_subcores - 1) * rpt
    assert rlast > 0
    mesh = plsc.VectorSubcoreMesh(core_axis_name="c", subcore_axis_name="s")
    zeros = jnp.zeros((n, C), jnp.float32)

    @functools.partial(
        pl.kernel, mesh=mesh,
        out_type=[jax.ShapeDtypeStruct((2, n, C), jnp.float32),
                  jax.ShapeDtypeStruct((2, n, C), jnp.float32)],
        scratch_types=[pltpu.VMEM((ch,), jnp.int32),
                       pltpu.VMEM((ch, C), jnp.float32),
                       pltpu.VMEM_SHARED((n, C), jnp.float32)],
    )
    def s1(dst_h, wv_h, wp_h, z_h, aggp_o, sp_o, idxv, rows, acc_sh):
        cid = lax.axis_index("c")
        sid = lax.axis_index("s")
        wid = sid * info.num_cores + cid
        base = wid * epw

        def phase(val_h, out_h):
            @pl.when(sid == 0)
            def _init():
                pltpu.sync_copy(z_h, acc_sh)

            plsc.subcore_barrier()

            def body(it, carry):
                off = base + it * ch
                pltpu.sync_copy(dst_h.at[pl.ds(off, ch)], idxv)
                pltpu.sync_copy(val_h.at[pl.ds(off, ch)], rows)
                pltpu.sync_copy(rows, acc_sh.at[idxv], add=True)
                return carry

            lax.fori_loop(0, nit, body, 0)
            plsc.subcore_barrier()

            @pl.when(sid < info.num_subcores - 1)
            def _copy_main():
                off = pl.multiple_of(sid * rpt, 8)
                pltpu.sync_copy(acc_sh.at[pl.ds(off, rpt)],
                                out_h.at[cid, pl.ds(off, rpt)])

            @pl.when(sid == info.num_subcores - 1)
            def _copy_tail():
                off = pl.multiple_of((info.num_subcores - 1) * rpt, 8)
                pltpu.sync_copy(acc_sh.at[pl.ds(off, rlast)],
                                out_h.at[cid, pl.ds(off, rlast)])

            plsc.subcore_barrier()

        phase(wv_h, aggp_o)
        phase(wp_h, sp_o)

    return s1(dst, wv, wp, zeros)


def kernel(x, pos, head, edges, params):
    a, m, t, c = x.shape
    n = a * m * t
    xf = jnp.transpose(x, (2, 1, 0, 3)).reshape(n, c)
    posf = jnp.transpose(pos, (2, 1, 0, 3)).reshape(n, 2)
    headf = jnp.transpose(head, (2, 1, 0)).reshape(n, 1)
    src = edges[0].astype(jnp.int32)
    dst = edges[1].astype(jnp.int32)
    fe, at = params['fe'], params['attn']

    tbl = jnp.concatenate(
        [posf, jnp.cos(headf), jnp.sin(headf), headf,
         jnp.zeros((n, 11), jnp.float32)], axis=-1)

    s16 = (jnp.arange(c)[:, None] // HEAD_DIM
           == jnp.arange(NUM_HEADS)[None, :]).astype(jnp.float32)
    s16t = s16.T

    xn, q, kv = _tc_n1(xf, tbl, at, bn=1008)
    qd, kvs = _sc_gather(src, dst, q, kv)
    sim, vj, bmax = _tc_e1(qd, kvs, fe, at, s16, be=1344)
    gmax = jnp.max(bmax, axis=0)
    wv, wp = _tc_e2(sim, vj, gmax, s16t, be=2688)
    aggp, sp = _sc_scatter(dst, wv, wp, n)
    out = _tc_n2(xf, xn, aggp, sp, at, s16t, bn=1008)
    return jnp.transpose(out.reshape(t, m, a, c), (2, 1, 0, 3))


# speedup vs baseline: 1.2152x; 1.0811x over previous
import functools
import math

import jax
import jax.numpy as jnp
from jax import lax
from jax.experimental import pallas as pl
from jax.experimental.pallas import tpu as pltpu
from jax.experimental.pallas import tpu_sc as plsc

C = 128
NUM_HEADS, HEAD_DIM = 8, 16


def _ln(xb, g, b, eps=1e-5):
    mu = jnp.mean(xb, axis=-1, keepdims=True)
    var = jnp.mean((xb - mu) ** 2, axis=-1, keepdims=True)
    return (xb - mu) / jnp.sqrt(var + eps) * g + b


def _n1_body(x_ref, tbl_ref, pg, pb, qwt, qb, kwt, vwt, vb,
             xn_ref, q_ref, kv_ref):
    xb = x_ref[...]
    tbl = tbl_ref[...]
    pad = jnp.zeros((xb.shape[0], 112), jnp.float32)
    xn = _ln(xb, pg[...], pb[...])
    xn_ref[...] = xn
    q = jnp.dot(xn, qwt[...], preferred_element_type=jnp.float32) + qb[...]
    q_ref[...] = jnp.concatenate([q, tbl, pad], axis=-1)
    k = jnp.dot(xn, kwt[...], preferred_element_type=jnp.float32)
    v = jnp.dot(xn, vwt[...], preferred_element_type=jnp.float32) + vb[...]
    kv_ref[...] = jnp.concatenate([k, v, tbl, pad], axis=-1)


_TWO_PI = 2.0 * math.pi
_INV_2PI = 1.0 / _TWO_PI
_SIN_C = (1.0, -1.0 / 6, 1.0 / 120, -1.0 / 5040, 1.0 / 362880,
          -1.0 / 39916800, 1.0 / 6227020800)
_COS_C = (1.0, -0.5, 1.0 / 24, -1.0 / 720, 1.0 / 40320, -1.0 / 3628800,
          1.0 / 479001600)


def _poly_even(y2, coefs):
    acc = jnp.full_like(y2, coefs[-1])
    for cc in coefs[-2::-1]:
        acc = acc * y2 + cc
    return acc


def _round_ne(x):
    return jnp.round(x)


def _e1_body(qd_ref, kv_ref,
             f2p, w1cc, w1cs, w1r, b1, g1, be1, w2t, b2,
             olng, olnb, outwt, outb, rg, rb, krt, vrt, vrb, s16,
             sim_ref, vj_ref, bmax_ref):
    qde = qd_ref[...]
    kve = kv_ref[...]
    ts = kve[:, 2 * C:2 * C + 16]
    td = qde[:, C:C + 16]
    relx = ts[:, 0:1] - td[:, 0:1]
    rely = ts[:, 1:2] - td[:, 1:2]
    d2 = relx * relx + rely * rely + 1e-12
    dist = d2 * lax.rsqrt(d2)
    cx = td[:, 2:3]
    cy = td[:, 3:4]
    direction = jnp.arctan2(cx * rely - cy * relx, cx * relx + cy * rely)
    ha = ts[:, 4:5] - td[:, 4:5]
    rh = ha - _TWO_PI * _round_ne(ha * _INV_2PI)
    rel = (dist, direction, rh)
    xfa = jnp.concatenate(
        [rel[i] * f2p[i:i + 1, :] for i in range(3)], axis=-1)
    y = xfa - _TWO_PI * _round_ne(xfa * _INV_2PI)
    y2 = y * y
    sina = y * _poly_even(y2, _SIN_C)
    cosa = _poly_even(y2, _COS_C)
    acc = jnp.zeros((qde.shape[0], C), jnp.float32)
    for i in range(3):
        h = (jnp.dot(cosa[:, 64 * i:64 * (i + 1)], w1cc[i],
                     preferred_element_type=jnp.float32)
             + jnp.dot(sina[:, 64 * i:64 * (i + 1)], w1cs[i],
                       preferred_element_type=jnp.float32)
             + rel[i] * w1r[i:i + 1, :] + b1[i:i + 1, :])
        h = _ln(h, g1[i:i + 1, :], be1[i:i + 1, :])
        h = jax.nn.relu(h)
        acc = acc + jnp.dot(h, w2t[i], preferred_element_type=jnp.float32) + b2[i:i + 1, :]
    r = jax.nn.relu(_ln(acc, olng[...], olnb[...]))
    r = jnp.dot(r, outwt[...], preferred_element_type=jnp.float32) + outb[...]
    rn = _ln(r, rg[...], rb[...])
    kj = kve[:, :C] + jnp.dot(rn, krt[...], preferred_element_type=jnp.float32)
    vj = (kve[:, C:2 * C]
          + jnp.dot(rn, vrt[...], preferred_element_type=jnp.float32) + vrb[...])
    sim = jnp.dot(qde[:, :C] * kj, s16[...],
                  preferred_element_type=jnp.float32) * (HEAD_DIM ** -0.5)
    sim_ref[...] = sim
    vj_ref[...] = vj
    bmax_ref[...] = jnp.max(sim, axis=0, keepdims=True)[None]


def _e2_body(sim_ref, vj_ref, gmax, s16t, wv_ref, wp_ref):
    w = jnp.exp(sim_ref[...] - gmax[...])
    wv_ref[...] = vj_ref[...] * jnp.dot(w, s16t[...],
                                        preferred_element_type=jnp.float32)
    wp_ref[...] = jnp.concatenate(
        [w, jnp.zeros((w.shape[0], C - NUM_HEADS), jnp.float32)], axis=-1)


def _n2_body(x_ref, xn_ref, aggp_ref, sp_ref, s16t,
             gwt1, gwt2, gb, swt, sb, owt, ob, postg, postb,
             ffpreg, ffpreb, ffw1t, ffb1, ffw2t, ffb2, ffpostg, ffpostb,
             out_ref):
    xb = x_ref[...]
    xn = xn_ref[...]
    ap = aggp_ref[...]
    sp = sp_ref[...]
    aggs = ap[0]
    ss = sp[0][:, 0:NUM_HEADS]
    for j in range(1, ap.shape[0]):
        aggs = aggs + ap[j]
        ss = ss + sp[j][:, 0:NUM_HEADS]
    recip = 1.0 / (ss + 1e-16)
    agg = aggs * jnp.dot(recip, s16t[...], preferred_element_type=jnp.float32)
    g = jax.nn.sigmoid(
        jnp.dot(agg, gwt1[...], preferred_element_type=jnp.float32)
        + jnp.dot(xn, gwt2[...], preferred_element_type=jnp.float32) + gb[...])
    sk = jnp.dot(xn, swt[...], preferred_element_type=jnp.float32) + sb[...]
    agg = agg + g * (sk - agg)
    out = jnp.dot(agg, owt[...], preferred_element_type=jnp.float32) + ob[...]
    x1 = xb + _ln(out, postg[...], postb[...])
    h = _ln(x1, ffpreg[...], ffpreb[...])
    h = jax.nn.relu(jnp.dot(h, ffw1t[...], preferred_element_type=jnp.float32)
                    + ffb1[...])
    h = jnp.dot(h, ffw2t[...], preferred_element_type=jnp.float32) + ffb2[...]
    out_ref[...] = x1 + _ln(h, ffpostg[...], ffpostb[...])


def _row(v):
    return v.reshape(1, -1)


def _pad_rows(a, rows=8):
    return jnp.pad(a, ((0, rows - a.shape[0]), (0, 0)))


def _full_spec(shape):
    nd = len(shape)
    return pl.BlockSpec(shape, lambda i, _nd=nd: (0,) * _nd)


def _tc_n1(xf, tbl, at, bn):
    n = xf.shape[0]
    grid = n // bn
    specs = [pl.BlockSpec((bn, C), lambda i: (i, 0)),
             pl.BlockSpec((bn, 16), lambda i: (i, 0))]
    wspecs = [_full_spec((1, C)), _full_spec((1, C)), _full_spec((C, C)),
              _full_spec((1, C)), _full_spec((C, C)), _full_spec((C, C)),
              _full_spec((1, C))]
    return pl.pallas_call(
        _n1_body,
        grid=(grid,),
        in_specs=specs + wspecs,
        out_specs=[pl.BlockSpec((bn, C), lambda i: (i, 0)),
                   pl.BlockSpec((bn, 2 * C), lambda i: (i, 0)),
                   pl.BlockSpec((bn, 3 * C), lambda i: (i, 0))],
        out_shape=[jax.ShapeDtypeStruct((n, C), jnp.float32),
                   jax.ShapeDtypeStruct((n, 2 * C), jnp.float32),
                   jax.ShapeDtypeStruct((n, 3 * C), jnp.float32)],
    )(xf, tbl, _row(at['pre_g']), _row(at['pre_b']), at['q_w'].T,
      _row(at['q_b']), at['k_w'].T, at['v_w'].T, _row(at['v_b']))


def _tc_e1(qd, kvs, fe, at, s16, be):
    e = qd.shape[0]
    grid = e // be
    f2p = _pad_rows(fe['freqs'] * (2 * math.pi))
    w1cc = jnp.transpose(fe['mlp_w1'][:, :, :64], (0, 2, 1))
    w1cs = jnp.transpose(fe['mlp_w1'][:, :, 64:C], (0, 2, 1))
    w1r = _pad_rows(fe['mlp_w1'][:, :, C])
    w2t = jnp.transpose(fe['mlp_w2'], (0, 2, 1))
    espec = lambda w: pl.BlockSpec((be, w), lambda i: (i, 0))
    in_specs = [espec(2 * C), espec(3 * C),
                _full_spec((8, 64)), _full_spec((3, 64, C)),
                _full_spec((3, 64, C)), _full_spec((8, C)),
                _full_spec((8, C)), _full_spec((8, C)), _full_spec((8, C)),
                _full_spec((3, C, C)), _full_spec((8, C)),
                _full_spec((1, C)), _full_spec((1, C)), _full_spec((C, C)),
                _full_spec((1, C)), _full_spec((1, C)), _full_spec((1, C)),
                _full_spec((C, C)), _full_spec((C, C)), _full_spec((1, C)),
                _full_spec((C, NUM_HEADS))]
    return pl.pallas_call(
        _e1_body,
        grid=(grid,),
        in_specs=in_specs,
        out_specs=[espec(NUM_HEADS), espec(C),
                   pl.BlockSpec((1, 1, NUM_HEADS), lambda i: (i, 0, 0))],
        out_shape=[jax.ShapeDtypeStruct((e, NUM_HEADS), jnp.float32),
                   jax.ShapeDtypeStruct((e, C), jnp.float32),
                   jax.ShapeDtypeStruct((grid, 1, NUM_HEADS), jnp.float32)],
    )(qd, kvs, f2p, w1cc, w1cs, w1r, _pad_rows(fe['mlp_b1']),
      _pad_rows(fe['mlp_ln_g']), _pad_rows(fe['mlp_ln_b']), w2t,
      _pad_rows(fe['mlp_b2']), _row(fe['out_ln_g']), _row(fe['out_ln_b']),
      fe['out_w'].T, _row(fe['out_b']), _row(at['r_g']), _row(at['r_b']),
      at['kr_w'].T, at['vr_w'].T, _row(at['vr_b']), s16)


def _tc_e2(sim, vj, gmax, s16t, be):
    e = sim.shape[0]
    grid = e // be
    return pl.pallas_call(
        _e2_body,
        grid=(grid,),
        in_specs=[pl.BlockSpec((be, NUM_HEADS), lambda i: (i, 0)),
                  pl.BlockSpec((be, C), lambda i: (i, 0)),
                  _full_spec((1, NUM_HEADS)), _full_spec((NUM_HEADS, C))],
        out_specs=[pl.BlockSpec((be, C), lambda i: (i, 0)),
                   pl.BlockSpec((be, C), lambda i: (i, 0))],
        out_shape=[jax.ShapeDtypeStruct((e, C), jnp.float32),
                   jax.ShapeDtypeStruct((e, C), jnp.float32)],
    )(sim, vj, gmax, s16t)


def _tc_n2(xf, xn, aggp, sp, at, s16t, bn):
    n = xf.shape[0]
    grid = n // bn
    nspec = lambda w: pl.BlockSpec((bn, w), lambda i: (i, 0))
    return pl.pallas_call(
        _n2_body,
        grid=(grid,),
        in_specs=[nspec(C), nspec(C),
                  pl.BlockSpec((aggp.shape[0], bn, C), lambda i: (0, i, 0)),
                  pl.BlockSpec((sp.shape[0], bn, C), lambda i: (0, i, 0)),
                  _full_spec((NUM_HEADS, C)),
                  _full_spec((C, C)), _full_spec((C, C)), _full_spec((1, C)),
                  _full_spec((C, C)), _full_spec((1, C)),
                  _full_spec((C, C)), _full_spec((1, C)),
                  _full_spec((1, C)), _full_spec((1, C)),
                  _full_spec((1, C)), _full_spec((1, C)),
                  _full_spec((C, 4 * C)), _full_spec((1, 4 * C)),
                  _full_spec((4 * C, C)), _full_spec((1, C)),
                  _full_spec((1, C)), _full_spec((1, C))],
        out_specs=[nspec(C)],
        out_shape=[jax.ShapeDtypeStruct((n, C), jnp.float32)],
    )(xf, xn, aggp, sp, s16t,
      at['g_w'][:, :C].T, at['g_w'][:, C:].T, _row(at['g_b']),
      at['s_w'].T, _row(at['s_b']), at['o_w'].T, _row(at['o_b']),
      _row(at['post_g']), _row(at['post_b']),
      _row(at['ffpre_g']), _row(at['ffpre_b']),
      at['ff_w1'].T, _row(at['ff_b1']), at['ff_w2'].T, _row(at['ff_b2']),
      _row(at['ffpost_g']), _row(at['ffpost_b']))[0]


def _pick_chunk(epw, cap):
    best = 8
    for c in range(8, cap + 1, 8):
        if epw % c == 0:
            best = c
    return best


def _sc_gather(src, dst, q, kv):
    e = src.shape[0]
    info = plsc.get_sparse_core_info()
    nw = info.num_cores * info.num_subcores
    epw = e // nw
    ch = _pick_chunk(epw, 144)
    nit = epw // ch
    mesh = plsc.VectorSubcoreMesh(core_axis_name="c", subcore_axis_name="s")

    @functools.partial(
        pl.kernel, mesh=mesh,
        out_type=[jax.ShapeDtypeStruct((e, 2 * C), jnp.float32),
                  jax.ShapeDtypeStruct((e, 3 * C), jnp.float32)],
        scratch_types=[pltpu.VMEM((ch,), jnp.int32),
                       pltpu.VMEM((ch,), jnp.int32),
                       pltpu.VMEM((ch, 2 * C), jnp.float32),
                       pltpu.VMEM((ch, 3 * C), jnp.float32),
                       pltpu.SemaphoreType.DMA],
    )
    def g1(src_h, dst_h, q_h, kv_h, qd_o, kvs_o, sidx, didx, qv, kvv, sem):
        wid = lax.axis_index("s") * info.num_cores + lax.axis_index("c")
        base = wid * epw

        def body(it, carry):
            off = base + it * ch
            pltpu.sync_copy(src_h.at[pl.ds(off, ch)], sidx)
            pltpu.sync_copy(dst_h.at[pl.ds(off, ch)], didx)
            c1 = pltpu.async_copy(q_h.at[didx], qv, sem)
            c2 = pltpu.async_copy(kv_h.at[sidx], kvv, sem)
            c1.wait()
            c2.wait()
            pltpu.sync_copy(qv, qd_o.at[pl.ds(off, ch)])
            pltpu.sync_copy(kvv, kvs_o.at[pl.ds(off, ch)])
            return carry

        lax.fori_loop(0, nit, body, 0)

    return g1(src, dst, q, kv)


def _sc_scatter(dst, wv, wp, n):
    e = dst.shape[0]
    info = plsc.get_sparse_core_info()
    nw = info.num_cores * info.num_subcores
    epw = e // nw
    ch = _pick_chunk(epw, 336)
    nit = epw // ch
    rpt = -(-n // info.num_subcores)
    rpt += (-rpt) % 8
    rlast = n - (info.num_subcores - 1) * rpt
    assert rlast > 0
    mesh = plsc.VectorSubcoreMesh(core_axis_name="c", subcore_axis_name="s")
    zeros = jnp.zeros((n, C), jnp.float32)

    @functools.partial(
        pl.kernel, mesh=mesh,
        out_type=[jax.ShapeDtypeStruct((2, n, C), jnp.float32),
                  jax.ShapeDtypeStruct((2, n, C), jnp.float32)],
        scratch_types=[pltpu.VMEM((ch,), jnp.int32),
                       pltpu.VMEM((ch, C), jnp.float32),
                       pltpu.VMEM_SHARED((n, C), jnp.float32)],
    )
    def s1(dst_h, wv_h, wp_h, z_h, aggp_o, sp_o, idxv, rows, acc_sh):
        cid = lax.axis_index("c")
        sid = lax.axis_index("s")
        wid = sid * info.num_cores + cid
        base = wid * epw

        def phase(val_h, out_h):
            @pl.when(sid == 0)
            def _init():
                pltpu.sync_copy(z_h, acc_sh)

            plsc.subcore_barrier()

            def body(it, carry):
                off = base + it * ch
                pltpu.sync_copy(dst_h.at[pl.ds(off, ch)], idxv)
                pltpu.sync_copy(val_h.at[pl.ds(off, ch)], rows)
                pltpu.sync_copy(rows, acc_sh.at[idxv], add=True)
                return carry

            lax.fori_loop(0, nit, body, 0)
            plsc.subcore_barrier()

            @pl.when(sid < info.num_subcores - 1)
            def _copy_main():
                off = pl.multiple_of(sid * rpt, 8)
                pltpu.sync_copy(acc_sh.at[pl.ds(off, rpt)],
                                out_h.at[cid, pl.ds(off, rpt)])

            @pl.when(sid == info.num_subcores - 1)
            def _copy_tail():
                off = pl.multiple_of((info.num_subcores - 1) * rpt, 8)
                pltpu.sync_copy(acc_sh.at[pl.ds(off, rlast)],
                                out_h.at[cid, pl.ds(off, rlast)])

            plsc.subcore_barrier()

        phase(wv_h, aggp_o)
        phase(wp_h, sp_o)

    return s1(dst, wv, wp, zeros)


def kernel(x, pos, head, edges, params):
    a, m, t, c = x.shape
    n = a * m * t
    xf = jnp.transpose(x, (2, 1, 0, 3)).reshape(n, c)
    posf = jnp.transpose(pos, (2, 1, 0, 3)).reshape(n, 2)
    headf = jnp.transpose(head, (2, 1, 0)).reshape(n, 1)
    src = edges[0].astype(jnp.int32)
    dst = edges[1].astype(jnp.int32)
    fe, at = params['fe'], params['attn']

    tbl = jnp.concatenate(
        [posf, jnp.cos(headf), jnp.sin(headf), headf,
         jnp.zeros((n, 11), jnp.float32)], axis=-1)

    s16 = (jnp.arange(c)[:, None] // HEAD_DIM
           == jnp.arange(NUM_HEADS)[None, :]).astype(jnp.float32)
    s16t = s16.T

    xn, q, kv = _tc_n1(xf, tbl, at, bn=1008)
    e = src.shape[0]
    parts = 2
    eh = e // parts
    halves = []
    for pi in range(parts):
        s_p = lax.slice(src, (pi * eh,), ((pi + 1) * eh,))
        d_p = lax.slice(dst, (pi * eh,), ((pi + 1) * eh,))
        qd_p, kvs_p = _sc_gather(s_p, d_p, q, kv)
        sim_p, vj_p, bmax_p = _tc_e1(qd_p, kvs_p, fe, at, s16, be=1344)
        halves.append((d_p, sim_p, vj_p, bmax_p))
    gmax = jnp.max(jnp.concatenate([h[3] for h in halves], axis=0), axis=0)
    aggps, sps = [], []
    for d_p, sim_p, vj_p, _ in halves:
        wv_p, wp_p = _tc_e2(sim_p, vj_p, gmax, s16t, be=2688)
        aggp_p, sp_p = _sc_scatter(d_p, wv_p, wp_p, n)
        aggps.append(aggp_p)
        sps.append(sp_p)
    aggp = jnp.concatenate(aggps, axis=0)
    sp = jnp.concatenate(sps, axis=0)
    out = _tc_n2(xf, xn, aggp, sp, at, s16t, bn=1008)
    return jnp.transpose(out.reshape(t, m, a, c), (2, 1, 0, 3))


# speedup vs baseline: 1.2186x; 1.0028x over previous
import functools
import math

import jax
import jax.numpy as jnp
from jax import lax
from jax.experimental import pallas as pl
from jax.experimental.pallas import tpu as pltpu
from jax.experimental.pallas import tpu_sc as plsc

C = 128
NUM_HEADS, HEAD_DIM = 8, 16


def _ln(xb, g, b, eps=1e-5):
    mu = jnp.mean(xb, axis=-1, keepdims=True)
    var = jnp.mean((xb - mu) ** 2, axis=-1, keepdims=True)
    return (xb - mu) / jnp.sqrt(var + eps) * g + b


def _n1_body(x_ref, tbl_ref, pg, pb, qwt, qb, kwt, vwt, vb,
             xn_ref, q_ref, kv_ref):
    xb = x_ref[...]
    tbl = tbl_ref[...]
    pad = jnp.zeros((xb.shape[0], 112), jnp.float32)
    xn = _ln(xb, pg[...], pb[...])
    xn_ref[...] = xn
    q = jnp.dot(xn, qwt[...], preferred_element_type=jnp.float32) + qb[...]
    q_ref[...] = jnp.concatenate([q, tbl, pad], axis=-1)
    k = jnp.dot(xn, kwt[...], preferred_element_type=jnp.float32)
    v = jnp.dot(xn, vwt[...], preferred_element_type=jnp.float32) + vb[...]
    kv_ref[...] = jnp.concatenate([k, v, tbl, pad], axis=-1)


_TWO_PI = 2.0 * math.pi
_INV_2PI = 1.0 / _TWO_PI
_SIN_C = (1.0, -1.0 / 6, 1.0 / 120, -1.0 / 5040, 1.0 / 362880,
          -1.0 / 39916800, 1.0 / 6227020800)
_COS_C = (1.0, -0.5, 1.0 / 24, -1.0 / 720, 1.0 / 40320, -1.0 / 3628800,
          1.0 / 479001600)


def _poly_even(y2, coefs):
    acc = jnp.full_like(y2, coefs[-1])
    for cc in coefs[-2::-1]:
        acc = acc * y2 + cc
    return acc


def _round_ne(x):
    return jnp.round(x)


def _e1_body(qd_ref, kv_ref,
             f2p, w1cc, w1cs, w1r, b1, g1, be1, w2t, b2,
             olng, olnb, outwt, outb, rg, rb, krt, vrt, vrb, s16,
             sim_ref, vj_ref, bmax_ref):
    qde = qd_ref[...]
    kve = kv_ref[...]
    ts = kve[:, 2 * C:2 * C + 16]
    td = qde[:, C:C + 16]
    relx = ts[:, 0:1] - td[:, 0:1]
    rely = ts[:, 1:2] - td[:, 1:2]
    d2 = relx * relx + rely * rely + 1e-12
    dist = d2 * lax.rsqrt(d2)
    cx = td[:, 2:3]
    cy = td[:, 3:4]
    direction = jnp.arctan2(cx * rely - cy * relx, cx * relx + cy * rely)
    ha = ts[:, 4:5] - td[:, 4:5]
    rh = ha - _TWO_PI * _round_ne(ha * _INV_2PI)
    rel = (dist, direction, rh)
    xfa = jnp.concatenate(
        [rel[i] * f2p[i:i + 1, :] for i in range(3)], axis=-1)
    y = xfa - _TWO_PI * _round_ne(xfa * _INV_2PI)
    y2 = y * y
    sina = y * _poly_even(y2, _SIN_C)
    cosa = _poly_even(y2, _COS_C)
    acc = jnp.zeros((qde.shape[0], C), jnp.float32)
    for i in range(3):
        h = (jnp.dot(cosa[:, 64 * i:64 * (i + 1)], w1cc[i],
                     preferred_element_type=jnp.float32)
             + jnp.dot(sina[:, 64 * i:64 * (i + 1)], w1cs[i],
                       preferred_element_type=jnp.float32)
             + rel[i] * w1r[i:i + 1, :] + b1[i:i + 1, :])
        h = _ln(h, g1[i:i + 1, :], be1[i:i + 1, :])
        h = jax.nn.relu(h)
        acc = acc + jnp.dot(h, w2t[i], preferred_element_type=jnp.float32) + b2[i:i + 1, :]
    r = jax.nn.relu(_ln(acc, olng[...], olnb[...]))
    r = jnp.dot(r, outwt[...], preferred_element_type=jnp.float32) + outb[...]
    rn = _ln(r, rg[...], rb[...])
    kj = kve[:, :C] + jnp.dot(rn, krt[...], preferred_element_type=jnp.float32)
    vj = (kve[:, C:2 * C]
          + jnp.dot(rn, vrt[...], preferred_element_type=jnp.float32) + vrb[...])
    sim = jnp.dot(qde[:, :C] * kj, s16[...],
                  preferred_element_type=jnp.float32) * (HEAD_DIM ** -0.5)
    sim_ref[...] = sim
    vj_ref[...] = vj
    bmax_ref[...] = jnp.max(sim, axis=0, keepdims=True)[None]


def _e2_body(sim_ref, vj_ref, gmax, s16t, wv_ref, wp_ref):
    w = jnp.exp(sim_ref[...] - gmax[...])
    wv_ref[...] = vj_ref[...] * jnp.dot(w, s16t[...],
                                        preferred_element_type=jnp.float32)
    wp_ref[...] = jnp.concatenate(
        [w, jnp.zeros((w.shape[0], C - NUM_HEADS), jnp.float32)], axis=-1)


def _n2_body(x_ref, xn_ref, aggp_ref, sp_ref, s16t,
             gwt1, gwt2, gb, swt, sb, owt, ob, postg, postb,
             ffpreg, ffpreb, ffw1t, ffb1, ffw2t, ffb2, ffpostg, ffpostb,
             out_ref):
    xb = x_ref[...]
    xn = xn_ref[...]
    ap = aggp_ref[...]
    sp = sp_ref[...]
    aggs = ap[0]
    ss = sp[0][:, 0:NUM_HEADS]
    for j in range(1, ap.shape[0]):
        aggs = aggs + ap[j]
        ss = ss + sp[j][:, 0:NUM_HEADS]
    recip = 1.0 / (ss + 1e-16)
    agg = aggs * jnp.dot(recip, s16t[...], preferred_element_type=jnp.float32)
    g = jax.nn.sigmoid(
        jnp.dot(agg, gwt1[...], preferred_element_type=jnp.float32)
        + jnp.dot(xn, gwt2[...], preferred_element_type=jnp.float32) + gb[...])
    sk = jnp.dot(xn, swt[...], preferred_element_type=jnp.float32) + sb[...]
    agg = agg + g * (sk - agg)
    out = jnp.dot(agg, owt[...], preferred_element_type=jnp.float32) + ob[...]
    x1 = xb + _ln(out, postg[...], postb[...])
    h = _ln(x1, ffpreg[...], ffpreb[...])
    h = jax.nn.relu(jnp.dot(h, ffw1t[...], preferred_element_type=jnp.float32)
                    + ffb1[...])
    h = jnp.dot(h, ffw2t[...], preferred_element_type=jnp.float32) + ffb2[...]
    out_ref[...] = x1 + _ln(h, ffpostg[...], ffpostb[...])


def _row(v):
    return v.reshape(1, -1)


def _pad_rows(a, rows=8):
    return jnp.pad(a, ((0, rows - a.shape[0]), (0, 0)))


def _full_spec(shape):
    nd = len(shape)
    return pl.BlockSpec(shape, lambda i, _nd=nd: (0,) * _nd)


def _tc_n1(xf, tbl, at, bn):
    n = xf.shape[0]
    grid = n // bn
    specs = [pl.BlockSpec((bn, C), lambda i: (i, 0)),
             pl.BlockSpec((bn, 16), lambda i: (i, 0))]
    wspecs = [_full_spec((1, C)), _full_spec((1, C)), _full_spec((C, C)),
              _full_spec((1, C)), _full_spec((C, C)), _full_spec((C, C)),
              _full_spec((1, C))]
    return pl.pallas_call(
        _n1_body,
        grid=(grid,),
        in_specs=specs + wspecs,
        out_specs=[pl.BlockSpec((bn, C), lambda i: (i, 0)),
                   pl.BlockSpec((bn, 2 * C), lambda i: (i, 0)),
                   pl.BlockSpec((bn, 3 * C), lambda i: (i, 0))],
        out_shape=[jax.ShapeDtypeStruct((n, C), jnp.float32),
                   jax.ShapeDtypeStruct((n, 2 * C), jnp.float32),
                   jax.ShapeDtypeStruct((n, 3 * C), jnp.float32)],
    )(xf, tbl, _row(at['pre_g']), _row(at['pre_b']), at['q_w'].T,
      _row(at['q_b']), at['k_w'].T, at['v_w'].T, _row(at['v_b']))


def _tc_e1(qd, kvs, fe, at, s16, be):
    e = qd.shape[0]
    grid = e // be
    f2p = _pad_rows(fe['freqs'] * (2 * math.pi))
    w1cc = jnp.transpose(fe['mlp_w1'][:, :, :64], (0, 2, 1))
    w1cs = jnp.transpose(fe['mlp_w1'][:, :, 64:C], (0, 2, 1))
    w1r = _pad_rows(fe['mlp_w1'][:, :, C])
    w2t = jnp.transpose(fe['mlp_w2'], (0, 2, 1))
    espec = lambda w: pl.BlockSpec((be, w), lambda i: (i, 0))
    in_specs = [espec(2 * C), espec(3 * C),
                _full_spec((8, 64)), _full_spec((3, 64, C)),
                _full_spec((3, 64, C)), _full_spec((8, C)),
                _full_spec((8, C)), _full_spec((8, C)), _full_spec((8, C)),
                _full_spec((3, C, C)), _full_spec((8, C)),
                _full_spec((1, C)), _full_spec((1, C)), _full_spec((C, C)),
                _full_spec((1, C)), _full_spec((1, C)), _full_spec((1, C)),
                _full_spec((C, C)), _full_spec((C, C)), _full_spec((1, C)),
                _full_spec((C, NUM_HEADS))]
    return pl.pallas_call(
        _e1_body,
        grid=(grid,),
        in_specs=in_specs,
        out_specs=[espec(NUM_HEADS), espec(C),
                   pl.BlockSpec((1, 1, NUM_HEADS), lambda i: (i, 0, 0))],
        out_shape=[jax.ShapeDtypeStruct((e, NUM_HEADS), jnp.float32),
                   jax.ShapeDtypeStruct((e, C), jnp.float32),
                   jax.ShapeDtypeStruct((grid, 1, NUM_HEADS), jnp.float32)],
    )(qd, kvs, f2p, w1cc, w1cs, w1r, _pad_rows(fe['mlp_b1']),
      _pad_rows(fe['mlp_ln_g']), _pad_rows(fe['mlp_ln_b']), w2t,
      _pad_rows(fe['mlp_b2']), _row(fe['out_ln_g']), _row(fe['out_ln_b']),
      fe['out_w'].T, _row(fe['out_b']), _row(at['r_g']), _row(at['r_b']),
      at['kr_w'].T, at['vr_w'].T, _row(at['vr_b']), s16)


def _tc_e2(sim, vj, gmax, s16t, be):
    e = sim.shape[0]
    grid = e // be
    return pl.pallas_call(
        _e2_body,
        grid=(grid,),
        in_specs=[pl.BlockSpec((be, NUM_HEADS), lambda i: (i, 0)),
                  pl.BlockSpec((be, C), lambda i: (i, 0)),
                  _full_spec((1, NUM_HEADS)), _full_spec((NUM_HEADS, C))],
        out_specs=[pl.BlockSpec((be, C), lambda i: (i, 0)),
                   pl.BlockSpec((be, C), lambda i: (i, 0))],
        out_shape=[jax.ShapeDtypeStruct((e, C), jnp.float32),
                   jax.ShapeDtypeStruct((e, C), jnp.float32)],
    )(sim, vj, gmax, s16t)


def _tc_n2(xf, xn, aggp, sp, at, s16t, bn):
    n = xf.shape[0]
    grid = n // bn
    nspec = lambda w: pl.BlockSpec((bn, w), lambda i: (i, 0))
    return pl.pallas_call(
        _n2_body,
        grid=(grid,),
        in_specs=[nspec(C), nspec(C),
                  pl.BlockSpec((aggp.shape[0], bn, C), lambda i: (0, i, 0)),
                  pl.BlockSpec((sp.shape[0], bn, C), lambda i: (0, i, 0)),
                  _full_spec((NUM_HEADS, C)),
                  _full_spec((C, C)), _full_spec((C, C)), _full_spec((1, C)),
                  _full_spec((C, C)), _full_spec((1, C)),
                  _full_spec((C, C)), _full_spec((1, C)),
                  _full_spec((1, C)), _full_spec((1, C)),
                  _full_spec((1, C)), _full_spec((1, C)),
                  _full_spec((C, 4 * C)), _full_spec((1, 4 * C)),
                  _full_spec((4 * C, C)), _full_spec((1, C)),
                  _full_spec((1, C)), _full_spec((1, C))],
        out_specs=[nspec(C)],
        out_shape=[jax.ShapeDtypeStruct((n, C), jnp.float32)],
    )(xf, xn, aggp, sp, s16t,
      at['g_w'][:, :C].T, at['g_w'][:, C:].T, _row(at['g_b']),
      at['s_w'].T, _row(at['s_b']), at['o_w'].T, _row(at['o_b']),
      _row(at['post_g']), _row(at['post_b']),
      _row(at['ffpre_g']), _row(at['ffpre_b']),
      at['ff_w1'].T, _row(at['ff_b1']), at['ff_w2'].T, _row(at['ff_b2']),
      _row(at['ffpost_g']), _row(at['ffpost_b']))[0]


def _pick_chunk(epw, cap):
    best = 8
    for c in range(8, cap + 1, 8):
        if epw % c == 0:
            best = c
    return best


def _sc_gather(src, dst, q, kv):
    e = src.shape[0]
    info = plsc.get_sparse_core_info()
    nw = info.num_cores * info.num_subcores
    epw = e // nw
    ch = _pick_chunk(epw, 144)
    assert epw % ch == 0
    nit = epw // ch
    mesh = plsc.VectorSubcoreMesh(core_axis_name="c", subcore_axis_name="s")

    @functools.partial(
        pl.kernel, mesh=mesh,
        out_type=[jax.ShapeDtypeStruct((e, 2 * C), jnp.float32),
                  jax.ShapeDtypeStruct((e, 3 * C), jnp.float32)],
        scratch_types=[pltpu.VMEM((ch,), jnp.int32),
                       pltpu.VMEM((ch,), jnp.int32),
                       pltpu.VMEM((ch, 2 * C), jnp.float32),
                       pltpu.VMEM((ch, 3 * C), jnp.float32),
                       pltpu.SemaphoreType.DMA],
    )
    def g1(src_h, dst_h, q_h, kv_h, qd_o, kvs_o, sidx, didx, qv, kvv, sem):
        wid = lax.axis_index("s") * info.num_cores + lax.axis_index("c")
        base = wid * epw

        def body(it, carry):
            off = base + it * ch
            pltpu.sync_copy(src_h.at[pl.ds(off, ch)], sidx)
            pltpu.sync_copy(dst_h.at[pl.ds(off, ch)], didx)
            c1 = pltpu.async_copy(q_h.at[didx], qv, sem)
            c2 = pltpu.async_copy(kv_h.at[sidx], kvv, sem)
            c1.wait()
            c2.wait()
            pltpu.sync_copy(qv, qd_o.at[pl.ds(off, ch)])
            pltpu.sync_copy(kvv, kvs_o.at[pl.ds(off, ch)])
            return carry

        lax.fori_loop(0, nit, body, 0)

    return g1(src, dst, q, kv)


def _sc_scatter(dst, wv, wp, n):
    e = dst.shape[0]
    info = plsc.get_sparse_core_info()
    nw = info.num_cores * info.num_subcores
    epw = e // nw
    ch = _pick_chunk(epw, 336)
    assert epw % ch == 0
    nit = epw // ch
    rpt = -(-n // info.num_subcores)
    rpt += (-rpt) % 8
    rlast = n - (info.num_subcores - 1) * rpt
    assert rlast > 0
    mesh = plsc.VectorSubcoreMesh(core_axis_name="c", subcore_axis_name="s")
    zeros = jnp.zeros((n, C), jnp.float32)

    @functools.partial(
        pl.kernel, mesh=mesh,
        out_type=[jax.ShapeDtypeStruct((2, n, C), jnp.float32),
                  jax.ShapeDtypeStruct((2, n, C), jnp.float32)],
        scratch_types=[pltpu.VMEM((ch,), jnp.int32),
                       pltpu.VMEM((ch, C), jnp.float32),
                       pltpu.VMEM_SHARED((n, C), jnp.float32)],
    )
    def s1(dst_h, wv_h, wp_h, z_h, aggp_o, sp_o, idxv, rows, acc_sh):
        cid = lax.axis_index("c")
        sid = lax.axis_index("s")
        wid = sid * info.num_cores + cid
        base = wid * epw

        def phase(val_h, out_h):
            @pl.when(sid == 0)
            def _init():
                pltpu.sync_copy(z_h, acc_sh)

            plsc.subcore_barrier()

            def body(it, carry):
                off = base + it * ch
                pltpu.sync_copy(dst_h.at[pl.ds(off, ch)], idxv)
                pltpu.sync_copy(val_h.at[pl.ds(off, ch)], rows)
                pltpu.sync_copy(rows, acc_sh.at[idxv], add=True)
                return carry

            lax.fori_loop(0, nit, body, 0)
            plsc.subcore_barrier()

            @pl.when(sid < info.num_subcores - 1)
            def _copy_main():
                off = pl.multiple_of(sid * rpt, 8)
                pltpu.sync_copy(acc_sh.at[pl.ds(off, rpt)],
                                out_h.at[cid, pl.ds(off, rpt)])

            @pl.when(sid == info.num_subcores - 1)
            def _copy_tail():
                off = pl.multiple_of((info.num_subcores - 1) * rpt, 8)
                pltpu.sync_copy(acc_sh.at[pl.ds(off, rlast)],
                                out_h.at[cid, pl.ds(off, rlast)])

            plsc.subcore_barrier()

        phase(wv_h, aggp_o)
        phase(wp_h, sp_o)

    return s1(dst, wv, wp, zeros)


def kernel(x, pos, head, edges, params):
    a, m, t, c = x.shape
    n = a * m * t
    xf = jnp.transpose(x, (2, 1, 0, 3)).reshape(n, c)
    posf = jnp.transpose(pos, (2, 1, 0, 3)).reshape(n, 2)
    headf = jnp.transpose(head, (2, 1, 0)).reshape(n, 1)
    src = edges[0].astype(jnp.int32)
    dst = edges[1].astype(jnp.int32)
    fe, at = params['fe'], params['attn']

    tbl = jnp.concatenate(
        [posf, jnp.cos(headf), jnp.sin(headf), headf,
         jnp.zeros((n, 11), jnp.float32)], axis=-1)

    s16 = (jnp.arange(c)[:, None] // HEAD_DIM
           == jnp.arange(NUM_HEADS)[None, :]).astype(jnp.float32)
    s16t = s16.T

    xn, q, kv = _tc_n1(xf, tbl, at, bn=1008)
    e = src.shape[0]
    parts = 3
    eh = e // parts
    halves = []
    for pi in range(parts):
        s_p = lax.slice(src, (pi * eh,), ((pi + 1) * eh,))
        d_p = lax.slice(dst, (pi * eh,), ((pi + 1) * eh,))
        qd_p, kvs_p = _sc_gather(s_p, d_p, q, kv)
        sim_p, vj_p, bmax_p = _tc_e1(qd_p, kvs_p, fe, at, s16, be=1344)
        halves.append((d_p, sim_p, vj_p, bmax_p))
    gmax = jnp.max(jnp.concatenate([h[3] for h in halves], axis=0), axis=0)
    aggps, sps = [], []
    for d_p, sim_p, vj_p, _ in halves:
        wv_p, wp_p = _tc_e2(sim_p, vj_p, gmax, s16t, be=2688)
        aggp_p, sp_p = _sc_scatter(d_p, wv_p, wp_p, n)
        aggps.append(aggp_p)
        sps.append(sp_p)
    aggp = jnp.concatenate(aggps, axis=0)
    sp = jnp.concatenate(sps, axis=0)
    out = _tc_n2(xf, xn, aggp, sp, at, s16t, bn=1008)
    return jnp.transpose(out.reshape(t, m, a, c), (2, 1, 0, 3))


# speedup vs baseline: 1.2664x; 1.0392x over previous
import functools
import math

import jax
import jax.numpy as jnp
from jax import lax
from jax.experimental import pallas as pl
from jax.experimental.pallas import tpu as pltpu
from jax.experimental.pallas import tpu_sc as plsc

C = 128
NUM_HEADS, HEAD_DIM = 8, 16


def _ln(xb, g, b, eps=1e-5):
    mu = jnp.mean(xb, axis=-1, keepdims=True)
    var = jnp.mean((xb - mu) ** 2, axis=-1, keepdims=True)
    return (xb - mu) * lax.rsqrt(var + eps) * g + b


def _n1_body(x_ref, tbl_ref, pg, pb, qwt, qb, kwt, vwt, vb,
             xn_ref, q_ref, kv_ref):
    xb = x_ref[...]
    tbl = tbl_ref[...]
    pad = jnp.zeros((xb.shape[0], 112), jnp.float32)
    xn = _ln(xb, pg[...], pb[...])
    xn_ref[...] = xn
    q = jnp.dot(xn, qwt[...], preferred_element_type=jnp.float32) + qb[...]
    q_ref[...] = jnp.concatenate([q, tbl, pad], axis=-1)
    k = jnp.dot(xn, kwt[...], preferred_element_type=jnp.float32)
    v = jnp.dot(xn, vwt[...], preferred_element_type=jnp.float32) + vb[...]
    kv_ref[...] = jnp.concatenate([k, v, tbl, pad], axis=-1)


_TWO_PI = 2.0 * math.pi
_INV_2PI = 1.0 / _TWO_PI
_SIN_C = (1.0, -1.0 / 6, 1.0 / 120, -1.0 / 5040, 1.0 / 362880,
          -1.0 / 39916800, 1.0 / 6227020800)
_COS_C = (1.0, -0.5, 1.0 / 24, -1.0 / 720, 1.0 / 40320, -1.0 / 3628800,
          1.0 / 479001600)


def _poly_even(y2, coefs):
    acc = jnp.full_like(y2, coefs[-1])
    for cc in coefs[-2::-1]:
        acc = acc * y2 + cc
    return acc


def _round_ne(x):
    return jnp.round(x)


def _e1_body(qd_ref, kv_ref,
             f2p, w1cc, w1cs, w1r, b1, g1, be1, w2t, b2,
             olng, olnb, outwt, outb, rg, rb, krt, vrt, vrb, s16,
             sim_ref, vj_ref, bmax_ref):
    qde = qd_ref[...]
    kve = kv_ref[...]
    ts = kve[:, 2 * C:2 * C + 16]
    td = qde[:, C:C + 16]
    relx = ts[:, 0:1] - td[:, 0:1]
    rely = ts[:, 1:2] - td[:, 1:2]
    d2 = relx * relx + rely * rely + 1e-12
    dist = d2 * lax.rsqrt(d2)
    cx = td[:, 2:3]
    cy = td[:, 3:4]
    direction = jnp.arctan2(cx * rely - cy * relx, cx * relx + cy * rely)
    ha = ts[:, 4:5] - td[:, 4:5]
    rh = ha - _TWO_PI * _round_ne(ha * _INV_2PI)
    rel = (dist, direction, rh)
    xfa = jnp.concatenate(
        [rel[i] * f2p[i:i + 1, :] for i in range(3)], axis=-1)
    y = xfa - _TWO_PI * _round_ne(xfa * _INV_2PI)
    y2 = y * y
    sina = y * _poly_even(y2, _SIN_C)
    cosa = _poly_even(y2, _COS_C)
    acc = jnp.zeros((qde.shape[0], C), jnp.float32)
    for i in range(3):
        h = (jnp.dot(cosa[:, 64 * i:64 * (i + 1)], w1cc[i],
                     preferred_element_type=jnp.float32)
             + jnp.dot(sina[:, 64 * i:64 * (i + 1)], w1cs[i],
                       preferred_element_type=jnp.float32)
             + rel[i] * w1r[i:i + 1, :] + b1[i:i + 1, :])
        h = _ln(h, g1[i:i + 1, :], be1[i:i + 1, :])
        h = jax.nn.relu(h)
        acc = acc + jnp.dot(h, w2t[i], preferred_element_type=jnp.float32) + b2[i:i + 1, :]
    r = jax.nn.relu(_ln(acc, olng[...], olnb[...]))
    r = jnp.dot(r, outwt[...], preferred_element_type=jnp.float32) + outb[...]
    rn = _ln(r, rg[...], rb[...])
    kj = kve[:, :C] + jnp.dot(rn, krt[...], preferred_element_type=jnp.float32)
    vj = (kve[:, C:2 * C]
          + jnp.dot(rn, vrt[...], preferred_element_type=jnp.float32) + vrb[...])
    sim = jnp.dot(qde[:, :C] * kj, s16[...],
                  preferred_element_type=jnp.float32) * (HEAD_DIM ** -0.5)
    sim_ref[...] = sim
    vj_ref[...] = vj
    bmax_ref[...] = jnp.max(sim, axis=0, keepdims=True)[None]


def _e2_body(sim_ref, vj_ref, gmax, s16t, wv_ref, wp_ref):
    w = jnp.exp(sim_ref[...] - gmax[...])
    wv_ref[...] = vj_ref[...] * jnp.dot(w, s16t[...],
                                        preferred_element_type=jnp.float32)
    wp_ref[...] = jnp.concatenate(
        [w, jnp.zeros((w.shape[0], C - NUM_HEADS), jnp.float32)], axis=-1)


def _n2_body(x_ref, xn_ref, aggp_ref, sp_ref, s16t,
             gwt1, gwt2, gb, swt, sb, owt, ob, postg, postb,
             ffpreg, ffpreb, ffw1t, ffb1, ffw2t, ffb2, ffpostg, ffpostb,
             out_ref):
    xb = x_ref[...]
    xn = xn_ref[...]
    ap = aggp_ref[...]
    sp = sp_ref[...]
    aggs = ap[0]
    ss = sp[0][:, 0:NUM_HEADS]
    for j in range(1, ap.shape[0]):
        aggs = aggs + ap[j]
        ss = ss + sp[j][:, 0:NUM_HEADS]
    recip = 1.0 / (ss + 1e-16)
    agg = aggs * jnp.dot(recip, s16t[...], preferred_element_type=jnp.float32)
    g = jax.nn.sigmoid(
        jnp.dot(agg, gwt1[...], preferred_element_type=jnp.float32)
        + jnp.dot(xn, gwt2[...], preferred_element_type=jnp.float32) + gb[...])
    sk = jnp.dot(xn, swt[...], preferred_element_type=jnp.float32) + sb[...]
    agg = agg + g * (sk - agg)
    out = jnp.dot(agg, owt[...], preferred_element_type=jnp.float32) + ob[...]
    x1 = xb + _ln(out, postg[...], postb[...])
    h = _ln(x1, ffpreg[...], ffpreb[...])
    h = jax.nn.relu(jnp.dot(h, ffw1t[...], preferred_element_type=jnp.float32)
                    + ffb1[...])
    h = jnp.dot(h, ffw2t[...], preferred_element_type=jnp.float32) + ffb2[...]
    out_ref[...] = x1 + _ln(h, ffpostg[...], ffpostb[...])


def _row(v):
    return v.reshape(1, -1)


def _pad_rows(a, rows=8):
    return jnp.pad(a, ((0, rows - a.shape[0]), (0, 0)))


def _full_spec(shape):
    nd = len(shape)
    return pl.BlockSpec(shape, lambda i, _nd=nd: (0,) * _nd)


def _tc_n1(xf, tbl, at, bn):
    n = xf.shape[0]
    grid = n // bn
    specs = [pl.BlockSpec((bn, C), lambda i: (i, 0)),
             pl.BlockSpec((bn, 16), lambda i: (i, 0))]
    wspecs = [_full_spec((1, C)), _full_spec((1, C)), _full_spec((C, C)),
              _full_spec((1, C)), _full_spec((C, C)), _full_spec((C, C)),
              _full_spec((1, C))]
    return pl.pallas_call(
        _n1_body,
        grid=(grid,),
        in_specs=specs + wspecs,
        out_specs=[pl.BlockSpec((bn, C), lambda i: (i, 0)),
                   pl.BlockSpec((bn, 2 * C), lambda i: (i, 0)),
                   pl.BlockSpec((bn, 3 * C), lambda i: (i, 0))],
        out_shape=[jax.ShapeDtypeStruct((n, C), jnp.float32),
                   jax.ShapeDtypeStruct((n, 2 * C), jnp.float32),
                   jax.ShapeDtypeStruct((n, 3 * C), jnp.float32)],
    )(xf, tbl, _row(at['pre_g']), _row(at['pre_b']), at['q_w'].T,
      _row(at['q_b']), at['k_w'].T, at['v_w'].T, _row(at['v_b']))


def _tc_e1(qd, kvs, fe, at, s16, be):
    e = qd.shape[0]
    grid = e // be
    f2p = _pad_rows(fe['freqs'] * (2 * math.pi))
    w1cc = jnp.transpose(fe['mlp_w1'][:, :, :64], (0, 2, 1))
    w1cs = jnp.transpose(fe['mlp_w1'][:, :, 64:C], (0, 2, 1))
    w1r = _pad_rows(fe['mlp_w1'][:, :, C])
    w2t = jnp.transpose(fe['mlp_w2'], (0, 2, 1))
    espec = lambda w: pl.BlockSpec((be, w), lambda i: (i, 0))
    in_specs = [espec(2 * C), espec(3 * C),
                _full_spec((8, 64)), _full_spec((3, 64, C)),
                _full_spec((3, 64, C)), _full_spec((8, C)),
                _full_spec((8, C)), _full_spec((8, C)), _full_spec((8, C)),
                _full_spec((3, C, C)), _full_spec((8, C)),
                _full_spec((1, C)), _full_spec((1, C)), _full_spec((C, C)),
                _full_spec((1, C)), _full_spec((1, C)), _full_spec((1, C)),
                _full_spec((C, C)), _full_spec((C, C)), _full_spec((1, C)),
                _full_spec((C, NUM_HEADS))]
    return pl.pallas_call(
        _e1_body,
        grid=(grid,),
        in_specs=in_specs,
        out_specs=[espec(NUM_HEADS), espec(C),
                   pl.BlockSpec((1, 1, NUM_HEADS), lambda i: (i, 0, 0))],
        out_shape=[jax.ShapeDtypeStruct((e, NUM_HEADS), jnp.float32),
                   jax.ShapeDtypeStruct((e, C), jnp.float32),
                   jax.ShapeDtypeStruct((grid, 1, NUM_HEADS), jnp.float32)],
    )(qd, kvs, f2p, w1cc, w1cs, w1r, _pad_rows(fe['mlp_b1']),
      _pad_rows(fe['mlp_ln_g']), _pad_rows(fe['mlp_ln_b']), w2t,
      _pad_rows(fe['mlp_b2']), _row(fe['out_ln_g']), _row(fe['out_ln_b']),
      fe['out_w'].T, _row(fe['out_b']), _row(at['r_g']), _row(at['r_b']),
      at['kr_w'].T, at['vr_w'].T, _row(at['vr_b']), s16)


def _tc_e2(sim, vj, gmax, s16t, be):
    e = sim.shape[0]
    grid = e // be
    return pl.pallas_call(
        _e2_body,
        grid=(grid,),
        in_specs=[pl.BlockSpec((be, NUM_HEADS), lambda i: (i, 0)),
                  pl.BlockSpec((be, C), lambda i: (i, 0)),
                  _full_spec((1, NUM_HEADS)), _full_spec((NUM_HEADS, C))],
        out_specs=[pl.BlockSpec((be, C), lambda i: (i, 0)),
                   pl.BlockSpec((be, C), lambda i: (i, 0))],
        out_shape=[jax.ShapeDtypeStruct((e, C), jnp.float32),
                   jax.ShapeDtypeStruct((e, C), jnp.float32)],
    )(sim, vj, gmax, s16t)


def _tc_n2(xf, xn, aggp, sp, at, s16t, bn):
    n = xf.shape[0]
    grid = n // bn
    nspec = lambda w: pl.BlockSpec((bn, w), lambda i: (i, 0))
    return pl.pallas_call(
        _n2_body,
        grid=(grid,),
        in_specs=[nspec(C), nspec(C),
                  pl.BlockSpec((aggp.shape[0], bn, C), lambda i: (0, i, 0)),
                  pl.BlockSpec((sp.shape[0], bn, C), lambda i: (0, i, 0)),
                  _full_spec((NUM_HEADS, C)),
                  _full_spec((C, C)), _full_spec((C, C)), _full_spec((1, C)),
                  _full_spec((C, C)), _full_spec((1, C)),
                  _full_spec((C, C)), _full_spec((1, C)),
                  _full_spec((1, C)), _full_spec((1, C)),
                  _full_spec((1, C)), _full_spec((1, C)),
                  _full_spec((C, 4 * C)), _full_spec((1, 4 * C)),
                  _full_spec((4 * C, C)), _full_spec((1, C)),
                  _full_spec((1, C)), _full_spec((1, C))],
        out_specs=[nspec(C)],
        out_shape=[jax.ShapeDtypeStruct((n, C), jnp.float32)],
    )(xf, xn, aggp, sp, s16t,
      at['g_w'][:, :C].T, at['g_w'][:, C:].T, _row(at['g_b']),
      at['s_w'].T, _row(at['s_b']), at['o_w'].T, _row(at['o_b']),
      _row(at['post_g']), _row(at['post_b']),
      _row(at['ffpre_g']), _row(at['ffpre_b']),
      at['ff_w1'].T, _row(at['ff_b1']), at['ff_w2'].T, _row(at['ff_b2']),
      _row(at['ffpost_g']), _row(at['ffpost_b']))[0]


def _pick_chunk(epw, cap):
    best = 8
    for c in range(8, cap + 1, 8):
        if epw % c == 0:
            best = c
    return best


def _sc_gather(src, dst, q, kv):
    e = src.shape[0]
    info = plsc.get_sparse_core_info()
    nw = info.num_cores * info.num_subcores
    epw = e // nw
    ch = _pick_chunk(epw, 144)
    assert epw % ch == 0
    nit = epw // ch
    mesh = plsc.VectorSubcoreMesh(core_axis_name="c", subcore_axis_name="s")

    @functools.partial(
        pl.kernel, mesh=mesh,
        out_type=[jax.ShapeDtypeStruct((e, 2 * C), jnp.float32),
                  jax.ShapeDtypeStruct((e, 3 * C), jnp.float32)],
        scratch_types=[pltpu.VMEM((ch,), jnp.int32),
                       pltpu.VMEM((ch,), jnp.int32),
                       pltpu.VMEM((ch, 2 * C), jnp.float32),
                       pltpu.VMEM((ch, 3 * C), jnp.float32),
                       pltpu.SemaphoreType.DMA],
    )
    def g1(src_h, dst_h, q_h, kv_h, qd_o, kvs_o, sidx, didx, qv, kvv, sem):
        wid = lax.axis_index("s") * info.num_cores + lax.axis_index("c")
        base = wid * epw

        def body(it, carry):
            off = base + it * ch
            pltpu.sync_copy(src_h.at[pl.ds(off, ch)], sidx)
            pltpu.sync_copy(dst_h.at[pl.ds(off, ch)], didx)
            c1 = pltpu.async_copy(q_h.at[didx], qv, sem)
            c2 = pltpu.async_copy(kv_h.at[sidx], kvv, sem)
            c1.wait()
            c2.wait()
            pltpu.sync_copy(qv, qd_o.at[pl.ds(off, ch)])
            pltpu.sync_copy(kvv, kvs_o.at[pl.ds(off, ch)])
            return carry

        lax.fori_loop(0, nit, body, 0)

    return g1(src, dst, q, kv)


def _sc_scatter(dst, wv, wp, n):
    e = dst.shape[0]
    info = plsc.get_sparse_core_info()
    nw = info.num_cores * info.num_subcores
    epw = e // nw
    ch = _pick_chunk(epw, 336)
    assert epw % ch == 0
    nit = epw // ch
    rpt = -(-n // info.num_subcores)
    rpt += (-rpt) % 8
    rlast = n - (info.num_subcores - 1) * rpt
    assert rlast > 0
    mesh = plsc.VectorSubcoreMesh(core_axis_name="c", subcore_axis_name="s")
    zeros = jnp.zeros((n, C), jnp.float32)

    @functools.partial(
        pl.kernel, mesh=mesh,
        out_type=[jax.ShapeDtypeStruct((2, n, C), jnp.float32),
                  jax.ShapeDtypeStruct((2, n, C), jnp.float32)],
        scratch_types=[pltpu.VMEM((ch,), jnp.int32),
                       pltpu.VMEM((ch, C), jnp.float32),
                       pltpu.VMEM_SHARED((n, C), jnp.float32)],
    )
    def s1(dst_h, wv_h, wp_h, z_h, aggp_o, sp_o, idxv, rows, acc_sh):
        cid = lax.axis_index("c")
        sid = lax.axis_index("s")
        wid = sid * info.num_cores + cid
        base = wid * epw

        def phase(val_h, out_h):
            @pl.when(sid == 0)
            def _init():
                pltpu.sync_copy(z_h, acc_sh)

            plsc.subcore_barrier()

            def body(it, carry):
                off = base + it * ch
                pltpu.sync_copy(dst_h.at[pl.ds(off, ch)], idxv)
                pltpu.sync_copy(val_h.at[pl.ds(off, ch)], rows)
                pltpu.sync_copy(rows, acc_sh.at[idxv], add=True)
                return carry

            lax.fori_loop(0, nit, body, 0)
            plsc.subcore_barrier()

            @pl.when(sid < info.num_subcores - 1)
            def _copy_main():
                off = pl.multiple_of(sid * rpt, 8)
                pltpu.sync_copy(acc_sh.at[pl.ds(off, rpt)],
                                out_h.at[cid, pl.ds(off, rpt)])

            @pl.when(sid == info.num_subcores - 1)
            def _copy_tail():
                off = pl.multiple_of((info.num_subcores - 1) * rpt, 8)
                pltpu.sync_copy(acc_sh.at[pl.ds(off, rlast)],
                                out_h.at[cid, pl.ds(off, rlast)])

            plsc.subcore_barrier()

        phase(wv_h, aggp_o)
        phase(wp_h, sp_o)

    return s1(dst, wv, wp, zeros)


def kernel(x, pos, head, edges, params):
    a, m, t, c = x.shape
    n = a * m * t
    xf = jnp.transpose(x, (2, 1, 0, 3)).reshape(n, c)
    posf = jnp.transpose(pos, (2, 1, 0, 3)).reshape(n, 2)
    headf = jnp.transpose(head, (2, 1, 0)).reshape(n, 1)
    src = edges[0].astype(jnp.int32)
    dst = edges[1].astype(jnp.int32)
    fe, at = params['fe'], params['attn']

    tbl = jnp.concatenate(
        [posf, jnp.cos(headf), jnp.sin(headf), headf,
         jnp.zeros((n, 11), jnp.float32)], axis=-1)

    s16 = (jnp.arange(c)[:, None] // HEAD_DIM
           == jnp.arange(NUM_HEADS)[None, :]).astype(jnp.float32)
    s16t = s16.T

    xn, q, kv = _tc_n1(xf, tbl, at, bn=1008)
    e = src.shape[0]
    parts = 3
    eh = e // parts
    halves = []
    for pi in range(parts):
        s_p = lax.slice(src, (pi * eh,), ((pi + 1) * eh,))
        d_p = lax.slice(dst, (pi * eh,), ((pi + 1) * eh,))
        qd_p, kvs_p = _sc_gather(s_p, d_p, q, kv)
        sim_p, vj_p, bmax_p = _tc_e1(qd_p, kvs_p, fe, at, s16, be=1344)
        halves.append((d_p, sim_p, vj_p, bmax_p))
    gmax = jnp.max(jnp.concatenate([h[3] for h in halves], axis=0), axis=0)
    aggps, sps = [], []
    for d_p, sim_p, vj_p, _ in halves:
        wv_p, wp_p = _tc_e2(sim_p, vj_p, gmax, s16t, be=2688)
        aggp_p, sp_p = _sc_scatter(d_p, wv_p, wp_p, n)
        aggps.append(aggp_p)
        sps.append(sp_p)
    aggp = jnp.concatenate(aggps, axis=0)
    sp = jnp.concatenate(sps, axis=0)
    out = _tc_n2(xf, xn, aggp, sp, at, s16t, bn=1008)
    return jnp.transpose(out.reshape(t, m, a, c), (2, 1, 0, 3))


# speedup vs baseline: 1.2801x; 1.0108x over previous
import functools
import math

import jax
import jax.numpy as jnp
from jax import lax
from jax.experimental import pallas as pl
from jax.experimental.pallas import tpu as pltpu
from jax.experimental.pallas import tpu_sc as plsc

C = 128
NUM_HEADS, HEAD_DIM = 8, 16


def _ln(xb, g, b, eps=1e-5):
    mu = jnp.mean(xb, axis=-1, keepdims=True)
    var = jnp.mean((xb - mu) ** 2, axis=-1, keepdims=True)
    return (xb - mu) * lax.rsqrt(var + eps) * g + b


def _n1_body(x_ref, tbl_ref, pg, pb, qwt, qb, kwt, vwt, vb,
             xn_ref, q_ref, kv_ref):
    xb = x_ref[...]
    tbl = tbl_ref[...]
    pad = jnp.zeros((xb.shape[0], 112), jnp.float32)
    xn = _ln(xb, pg[...], pb[...])
    xn_ref[...] = xn
    q = jnp.dot(xn, qwt[...], preferred_element_type=jnp.float32) + qb[...]
    q_ref[...] = jnp.concatenate([q, tbl, pad], axis=-1)
    k = jnp.dot(xn, kwt[...], preferred_element_type=jnp.float32)
    v = jnp.dot(xn, vwt[...], preferred_element_type=jnp.float32) + vb[...]
    kv_ref[...] = jnp.concatenate([k, v, tbl, pad], axis=-1)


_TWO_PI = 2.0 * math.pi
_INV_2PI = 1.0 / _TWO_PI
_SIN_C = (1.0, -1.0 / 6, 1.0 / 120, -1.0 / 5040, 1.0 / 362880,
          -1.0 / 39916800, 1.0 / 6227020800)
_COS_C = (1.0, -0.5, 1.0 / 24, -1.0 / 720, 1.0 / 40320, -1.0 / 3628800,
          1.0 / 479001600)


def _poly_even(y2, coefs):
    acc = jnp.full_like(y2, coefs[-1])
    for cc in coefs[-2::-1]:
        acc = acc * y2 + cc
    return acc


def _round_ne(x):
    return jnp.round(x)


def _e1_body(qd_ref, kv_ref,
             f2p, w1cc, w1cs, w1r, b1, g1, be1, w2t, b2,
             olng, olnb, outwt, outb, rg, rb, krt, vrt, vrb, s16,
             sim_ref, vj_ref, bmax_ref):
    qde = qd_ref[...]
    kve = kv_ref[...]
    ts = kve[:, 2 * C:2 * C + 16]
    td = qde[:, C:C + 16]
    relx = ts[:, 0:1] - td[:, 0:1]
    rely = ts[:, 1:2] - td[:, 1:2]
    d2 = relx * relx + rely * rely + 1e-12
    dist = d2 * lax.rsqrt(d2)
    cx = td[:, 2:3]
    cy = td[:, 3:4]
    direction = jnp.arctan2(cx * rely - cy * relx, cx * relx + cy * rely)
    ha = ts[:, 4:5] - td[:, 4:5]
    rh = ha - _TWO_PI * _round_ne(ha * _INV_2PI)
    rel = (dist, direction, rh)
    xfa = jnp.concatenate(
        [rel[i] * f2p[i:i + 1, :] for i in range(3)], axis=-1)
    y = xfa - _TWO_PI * _round_ne(xfa * _INV_2PI)
    y2 = y * y
    sina = y * _poly_even(y2, _SIN_C)
    cosa = _poly_even(y2, _COS_C)
    acc = jnp.zeros((qde.shape[0], C), jnp.float32)
    for i in range(3):
        h = (jnp.dot(cosa[:, 64 * i:64 * (i + 1)], w1cc[i],
                     preferred_element_type=jnp.float32)
             + jnp.dot(sina[:, 64 * i:64 * (i + 1)], w1cs[i],
                       preferred_element_type=jnp.float32)
             + rel[i] * w1r[i:i + 1, :] + b1[i:i + 1, :])
        h = _ln(h, g1[i:i + 1, :], be1[i:i + 1, :])
        h = jax.nn.relu(h)
        acc = acc + jnp.dot(h, w2t[i], preferred_element_type=jnp.float32) + b2[i:i + 1, :]
    r = jax.nn.relu(_ln(acc, olng[...], olnb[...]))
    r = jnp.dot(r, outwt[...], preferred_element_type=jnp.float32) + outb[...]
    rn = _ln(r, rg[...], rb[...])
    kj = kve[:, :C] + jnp.dot(rn, krt[...], preferred_element_type=jnp.float32)
    vj = (kve[:, C:2 * C]
          + jnp.dot(rn, vrt[...], preferred_element_type=jnp.float32) + vrb[...])
    sim = jnp.dot(qde[:, :C] * kj, s16[...],
                  preferred_element_type=jnp.float32) * (HEAD_DIM ** -0.5)
    sim_ref[...] = sim
    vj_ref[...] = vj
    bmax_ref[...] = jnp.max(sim, axis=0, keepdims=True)[None]


def _e2_body(sim_ref, vj_ref, gmax, s16t, wv_ref, wp_ref):
    w = jnp.exp(sim_ref[...] - gmax[...])
    wv_ref[...] = vj_ref[...] * jnp.dot(w, s16t[...],
                                        preferred_element_type=jnp.float32)
    wp_ref[...] = jnp.concatenate(
        [w, jnp.zeros((w.shape[0], C - NUM_HEADS), jnp.float32)], axis=-1)


def _n2_body(x_ref, xn_ref, aggp_ref, sp_ref, s16t,
             gwt1, gwt2, gb, swt, sb, owt, ob, postg, postb,
             ffpreg, ffpreb, ffw1t, ffb1, ffw2t, ffb2, ffpostg, ffpostb,
             out_ref):
    xb = x_ref[...]
    xn = xn_ref[...]
    ap = aggp_ref[...]
    sp = sp_ref[...]
    aggs = ap[0]
    ss = sp[0][:, 0:NUM_HEADS]
    for j in range(1, ap.shape[0]):
        aggs = aggs + ap[j]
        ss = ss + sp[j][:, 0:NUM_HEADS]
    recip = 1.0 / (ss + 1e-16)
    agg = aggs * jnp.dot(recip, s16t[...], preferred_element_type=jnp.float32)
    g = jax.nn.sigmoid(
        jnp.dot(agg, gwt1[...], preferred_element_type=jnp.float32)
        + jnp.dot(xn, gwt2[...], preferred_element_type=jnp.float32) + gb[...])
    sk = jnp.dot(xn, swt[...], preferred_element_type=jnp.float32) + sb[...]
    agg = agg + g * (sk - agg)
    out = jnp.dot(agg, owt[...], preferred_element_type=jnp.float32) + ob[...]
    x1 = xb + _ln(out, postg[...], postb[...])
    h = _ln(x1, ffpreg[...], ffpreb[...])
    h = jax.nn.relu(jnp.dot(h, ffw1t[...], preferred_element_type=jnp.float32)
                    + ffb1[...])
    h = jnp.dot(h, ffw2t[...], preferred_element_type=jnp.float32) + ffb2[...]
    out_ref[...] = x1 + _ln(h, ffpostg[...], ffpostb[...])


def _row(v):
    return v.reshape(1, -1)


def _pad_rows(a, rows=8):
    return jnp.pad(a, ((0, rows - a.shape[0]), (0, 0)))


def _full_spec(shape):
    nd = len(shape)
    return pl.BlockSpec(shape, lambda i, _nd=nd: (0,) * _nd)


def _tc_n1(xf, tbl, at, bn):
    n = xf.shape[0]
    grid = n // bn
    specs = [pl.BlockSpec((bn, C), lambda i: (i, 0)),
             pl.BlockSpec((bn, 16), lambda i: (i, 0))]
    wspecs = [_full_spec((1, C)), _full_spec((1, C)), _full_spec((C, C)),
              _full_spec((1, C)), _full_spec((C, C)), _full_spec((C, C)),
              _full_spec((1, C))]
    return pl.pallas_call(
        _n1_body,
        grid=(grid,),
        in_specs=specs + wspecs,
        out_specs=[pl.BlockSpec((bn, C), lambda i: (i, 0)),
                   pl.BlockSpec((bn, 2 * C), lambda i: (i, 0)),
                   pl.BlockSpec((bn, 3 * C), lambda i: (i, 0))],
        out_shape=[jax.ShapeDtypeStruct((n, C), jnp.float32),
                   jax.ShapeDtypeStruct((n, 2 * C), jnp.float32),
                   jax.ShapeDtypeStruct((n, 3 * C), jnp.float32)],
    )(xf, tbl, _row(at['pre_g']), _row(at['pre_b']), at['q_w'].T,
      _row(at['q_b']), at['k_w'].T, at['v_w'].T, _row(at['v_b']))


def _tc_e1(qd, kvs, fe, at, s16, be):
    e = qd.shape[0]
    grid = e // be
    f2p = _pad_rows(fe['freqs'] * (2 * math.pi))
    w1cc = jnp.transpose(fe['mlp_w1'][:, :, :64], (0, 2, 1))
    w1cs = jnp.transpose(fe['mlp_w1'][:, :, 64:C], (0, 2, 1))
    w1r = _pad_rows(fe['mlp_w1'][:, :, C])
    w2t = jnp.transpose(fe['mlp_w2'], (0, 2, 1))
    espec = lambda w: pl.BlockSpec((be, w), lambda i: (i, 0))
    in_specs = [espec(2 * C), espec(3 * C),
                _full_spec((8, 64)), _full_spec((3, 64, C)),
                _full_spec((3, 64, C)), _full_spec((8, C)),
                _full_spec((8, C)), _full_spec((8, C)), _full_spec((8, C)),
                _full_spec((3, C, C)), _full_spec((8, C)),
                _full_spec((1, C)), _full_spec((1, C)), _full_spec((C, C)),
                _full_spec((1, C)), _full_spec((1, C)), _full_spec((1, C)),
                _full_spec((C, C)), _full_spec((C, C)), _full_spec((1, C)),
                _full_spec((C, NUM_HEADS))]
    return pl.pallas_call(
        _e1_body,
        grid=(grid,),
        in_specs=in_specs,
        out_specs=[espec(NUM_HEADS), espec(C),
                   pl.BlockSpec((1, 1, NUM_HEADS), lambda i: (i, 0, 0))],
        out_shape=[jax.ShapeDtypeStruct((e, NUM_HEADS), jnp.float32),
                   jax.ShapeDtypeStruct((e, C), jnp.float32),
                   jax.ShapeDtypeStruct((grid, 1, NUM_HEADS), jnp.float32)],
    )(qd, kvs, f2p, w1cc, w1cs, w1r, _pad_rows(fe['mlp_b1']),
      _pad_rows(fe['mlp_ln_g']), _pad_rows(fe['mlp_ln_b']), w2t,
      _pad_rows(fe['mlp_b2']), _row(fe['out_ln_g']), _row(fe['out_ln_b']),
      fe['out_w'].T, _row(fe['out_b']), _row(at['r_g']), _row(at['r_b']),
      at['kr_w'].T, at['vr_w'].T, _row(at['vr_b']), s16)


def _tc_e2(sim, vj, gmax, s16t, be):
    e = sim.shape[0]
    grid = e // be
    return pl.pallas_call(
        _e2_body,
        grid=(grid,),
        in_specs=[pl.BlockSpec((be, NUM_HEADS), lambda i: (i, 0)),
                  pl.BlockSpec((be, C), lambda i: (i, 0)),
                  _full_spec((1, NUM_HEADS)), _full_spec((NUM_HEADS, C))],
        out_specs=[pl.BlockSpec((be, C), lambda i: (i, 0)),
                   pl.BlockSpec((be, C), lambda i: (i, 0))],
        out_shape=[jax.ShapeDtypeStruct((e, C), jnp.float32),
                   jax.ShapeDtypeStruct((e, C), jnp.float32)],
    )(sim, vj, gmax, s16t)


def _tc_n2(xf, xn, aggp, sp, at, s16t, bn):
    n = xf.shape[0]
    grid = n // bn
    nspec = lambda w: pl.BlockSpec((bn, w), lambda i: (i, 0))
    return pl.pallas_call(
        _n2_body,
        grid=(grid,),
        in_specs=[nspec(C), nspec(C),
                  pl.BlockSpec((aggp.shape[0], bn, C), lambda i: (0, i, 0)),
                  pl.BlockSpec((sp.shape[0], bn, C), lambda i: (0, i, 0)),
                  _full_spec((NUM_HEADS, C)),
                  _full_spec((C, C)), _full_spec((C, C)), _full_spec((1, C)),
                  _full_spec((C, C)), _full_spec((1, C)),
                  _full_spec((C, C)), _full_spec((1, C)),
                  _full_spec((1, C)), _full_spec((1, C)),
                  _full_spec((1, C)), _full_spec((1, C)),
                  _full_spec((C, 4 * C)), _full_spec((1, 4 * C)),
                  _full_spec((4 * C, C)), _full_spec((1, C)),
                  _full_spec((1, C)), _full_spec((1, C))],
        out_specs=[nspec(C)],
        out_shape=[jax.ShapeDtypeStruct((n, C), jnp.float32)],
    )(xf, xn, aggp, sp, s16t,
      at['g_w'][:, :C].T, at['g_w'][:, C:].T, _row(at['g_b']),
      at['s_w'].T, _row(at['s_b']), at['o_w'].T, _row(at['o_b']),
      _row(at['post_g']), _row(at['post_b']),
      _row(at['ffpre_g']), _row(at['ffpre_b']),
      at['ff_w1'].T, _row(at['ff_b1']), at['ff_w2'].T, _row(at['ff_b2']),
      _row(at['ffpost_g']), _row(at['ffpost_b']))[0]


def _pick_chunk(epw, cap):
    best = 8
    for c in range(8, cap + 1, 8):
        if epw % c == 0:
            best = c
    return best


def _sc_gather(src, dst, q, kv):
    e = src.shape[0]
    info = plsc.get_sparse_core_info()
    nw = info.num_cores * info.num_subcores
    epw = e // nw
    ch = _pick_chunk(epw, 144)
    assert epw % ch == 0
    nit = epw // ch
    mesh = plsc.VectorSubcoreMesh(core_axis_name="c", subcore_axis_name="s")

    @functools.partial(
        pl.kernel, mesh=mesh,
        out_type=[jax.ShapeDtypeStruct((e, 2 * C), jnp.float32),
                  jax.ShapeDtypeStruct((e, 3 * C), jnp.float32)],
        scratch_types=[pltpu.VMEM((ch,), jnp.int32),
                       pltpu.VMEM((ch,), jnp.int32),
                       pltpu.VMEM((ch, 2 * C), jnp.float32),
                       pltpu.VMEM((ch, 3 * C), jnp.float32),
                       pltpu.SemaphoreType.DMA],
    )
    def g1(src_h, dst_h, q_h, kv_h, qd_o, kvs_o, sidx, didx, qv, kvv, sem):
        wid = lax.axis_index("s") * info.num_cores + lax.axis_index("c")
        base = wid * epw

        def body(it, carry):
            off = base + it * ch
            pltpu.sync_copy(src_h.at[pl.ds(off, ch)], sidx)
            pltpu.sync_copy(dst_h.at[pl.ds(off, ch)], didx)
            c1 = pltpu.async_copy(q_h.at[didx], qv, sem)
            c2 = pltpu.async_copy(kv_h.at[sidx], kvv, sem)
            c1.wait()
            c2.wait()
            pltpu.sync_copy(qv, qd_o.at[pl.ds(off, ch)])
            pltpu.sync_copy(kvv, kvs_o.at[pl.ds(off, ch)])
            return carry

        lax.fori_loop(0, nit, body, 0)

    return g1(src, dst, q, kv)


def _sc_scatter(dst, wv, wp, n):
    e = dst.shape[0]
    info = plsc.get_sparse_core_info()
    nw = info.num_cores * info.num_subcores
    epw = e // nw
    ch = _pick_chunk(epw, 336)
    assert epw % ch == 0
    nit = epw // ch
    rpt = -(-n // info.num_subcores)
    rpt += (-rpt) % 8
    rlast = n - (info.num_subcores - 1) * rpt
    assert rlast > 0
    mesh = plsc.VectorSubcoreMesh(core_axis_name="c", subcore_axis_name="s")
    zeros = jnp.zeros((n, C), jnp.float32)

    @functools.partial(
        pl.kernel, mesh=mesh,
        out_type=[jax.ShapeDtypeStruct((2, n, C), jnp.float32),
                  jax.ShapeDtypeStruct((2, n, C), jnp.float32)],
        scratch_types=[pltpu.VMEM((ch,), jnp.int32),
                       pltpu.VMEM((ch, C), jnp.float32),
                       pltpu.VMEM_SHARED((n, C), jnp.float32)],
    )
    def s1(dst_h, wv_h, wp_h, z_h, aggp_o, sp_o, idxv, rows, acc_sh):
        cid = lax.axis_index("c")
        sid = lax.axis_index("s")
        wid = sid * info.num_cores + cid
        base = wid * epw

        def phase(val_h, out_h):
            @pl.when(sid == 0)
            def _init():
                pltpu.sync_copy(z_h, acc_sh)

            plsc.subcore_barrier()

            def body(it, carry):
                off = base + it * ch
                pltpu.sync_copy(dst_h.at[pl.ds(off, ch)], idxv)
                pltpu.sync_copy(val_h.at[pl.ds(off, ch)], rows)
                pltpu.sync_copy(rows, acc_sh.at[idxv], add=True)
                return carry

            lax.fori_loop(0, nit, body, 0)
            plsc.subcore_barrier()

            @pl.when(sid < info.num_subcores - 1)
            def _copy_main():
                off = pl.multiple_of(sid * rpt, 8)
                pltpu.sync_copy(acc_sh.at[pl.ds(off, rpt)],
                                out_h.at[cid, pl.ds(off, rpt)])

            @pl.when(sid == info.num_subcores - 1)
            def _copy_tail():
                off = pl.multiple_of((info.num_subcores - 1) * rpt, 8)
                pltpu.sync_copy(acc_sh.at[pl.ds(off, rlast)],
                                out_h.at[cid, pl.ds(off, rlast)])

            plsc.subcore_barrier()

        phase(wv_h, aggp_o)
        phase(wp_h, sp_o)

    return s1(dst, wv, wp, zeros)


def kernel(x, pos, head, edges, params):
    a, m, t, c = x.shape
    n = a * m * t
    xf = jnp.transpose(x, (2, 1, 0, 3)).reshape(n, c)
    posf = jnp.transpose(pos, (2, 1, 0, 3)).reshape(n, 2)
    headf = jnp.transpose(head, (2, 1, 0)).reshape(n, 1)
    src = edges[0].astype(jnp.int32)
    dst = edges[1].astype(jnp.int32)
    fe, at = params['fe'], params['attn']

    tbl = jnp.concatenate(
        [posf, jnp.cos(headf), jnp.sin(headf), headf,
         jnp.zeros((n, 11), jnp.float32)], axis=-1)

    s16 = (jnp.arange(c)[:, None] // HEAD_DIM
           == jnp.arange(NUM_HEADS)[None, :]).astype(jnp.float32)
    s16t = s16.T

    xn, q, kv = _tc_n1(xf, tbl, at, bn=2016)
    e = src.shape[0]
    parts = 3
    eh = e // parts
    halves = []
    for pi in range(parts):
        s_p = lax.slice(src, (pi * eh,), ((pi + 1) * eh,))
        d_p = lax.slice(dst, (pi * eh,), ((pi + 1) * eh,))
        qd_p, kvs_p = _sc_gather(s_p, d_p, q, kv)
        sim_p, vj_p, bmax_p = _tc_e1(qd_p, kvs_p, fe, at, s16, be=2688)
        halves.append((d_p, sim_p, vj_p, bmax_p))
    gmax = jnp.max(jnp.concatenate([h[3] for h in halves], axis=0), axis=0)
    aggps, sps = [], []
    for d_p, sim_p, vj_p, _ in halves:
        wv_p, wp_p = _tc_e2(sim_p, vj_p, gmax, s16t, be=5376)
        aggp_p, sp_p = _sc_scatter(d_p, wv_p, wp_p, n)
        aggps.append(aggp_p)
        sps.append(sp_p)
    aggp = jnp.concatenate(aggps, axis=0)
    sp = jnp.concatenate(sps, axis=0)
    out = _tc_n2(xf, xn, aggp, sp, at, s16t, bn=2016)
    return jnp.transpose(out.reshape(t, m, a, c), (2, 1, 0, 3))
